# Initial kernel scaffold; baseline (speedup 1.0000x reference)
import functools

import jax
import jax.numpy as jnp
from jax import lax
from jax.experimental import pallas as pl
from jax.experimental.pallas import tpu as pltpu
from jax.experimental.pallas import tpu_sc as plsc

N_DRUG = 10000
N_PROT = 50000
D = 128
OUT = 64
E = 320000
L = 320000

HALF = 25024
ACC_ROWS = 25088
TRASH = 25080
NPROT_PAD = 2 * HALF

f32 = jnp.float32
i32 = jnp.int32


def _proj_drug_body(x_ref, wl_ref, wlin_ref, z_ref, dp_ref):
    w2 = wlin_ref[D:, :]
    wl2 = jnp.dot(wl_ref[:], w2, preferred_element_type=f32)
    x = x_ref[:]
    z_ref[:] = jnp.dot(x, wl2, preferred_element_type=f32)
    dp_ref[:] = jnp.dot(x, wlin_ref[:D, :], preferred_element_type=f32)


def _proj_drug(x_drug, W_l, W_lin):
    blk = 1000
    grid = N_DRUG // blk
    return pl.pallas_call(
        _proj_drug_body,
        grid=(grid,),
        in_specs=[
            pl.BlockSpec((blk, D), lambda i: (i, 0)),
            pl.BlockSpec((D, D), lambda i: (0, 0)),
            pl.BlockSpec((2 * D, OUT), lambda i: (0, 0)),
        ],
        out_specs=[
            pl.BlockSpec((blk, OUT), lambda i: (i, 0)),
            pl.BlockSpec((blk, OUT), lambda i: (i, 0)),
        ],
        out_shape=[
            jax.ShapeDtypeStruct((N_DRUG, OUT), f32),
            jax.ShapeDtypeStruct((N_DRUG, OUT), f32),
        ],
    )(x_drug, W_l, W_lin)


def _prot_body(sum_ref, cnt_ref, xp_ref, wr_ref, wlin_ref, bl_ref, blin_ref, out_ref):
    w2 = wlin_ref[D:, :]
    wr2 = jnp.dot(wr_ref[:], w2, preferred_element_type=f32)
    b2 = jnp.dot(bl_ref[:], w2, preferred_element_type=f32) + blin_ref[:]
    cnt = cnt_ref[:, 0:1]
    inv = 1.0 / jnp.maximum(cnt, 1.0)
    out_ref[:] = (
        sum_ref[:] * inv
        + jnp.dot(xp_ref[:], wr2, preferred_element_type=f32)
        + b2
    )


def _proj_prot(summed, cnt16, x_protein, W_r, W_lin, b_l, b_lin):
    blk = 500
    grid = N_PROT // blk
    return pl.pallas_call(
        _prot_body,
        grid=(grid,),
        in_specs=[
            pl.BlockSpec((blk, OUT), lambda i: (i, 0)),
            pl.BlockSpec((blk, 16), lambda i: (i, 0)),
            pl.BlockSpec((blk, D), lambda i: (i, 0)),
            pl.BlockSpec((D, D), lambda i: (0, 0)),
            pl.BlockSpec((2 * D, OUT), lambda i: (0, 0)),
            pl.BlockSpec((1, D), lambda i: (0, 0)),
            pl.BlockSpec((1, OUT), lambda i: (0, 0)),
        ],
        out_specs=pl.BlockSpec((blk, OUT), lambda i: (i, 0)),
        out_shape=jax.ShapeDtypeStruct((N_PROT, OUT), f32),
    )(summed, cnt16, x_protein, W_r, W_lin, b_l, b_lin)


_MESH = dict(core_axis_name="c", subcore_axis_name="s")

_B_BATCHES = E // 128
_B_PER_TILE = _B_BATCHES // 16
_B_LEFT = _B_BATCHES - 16 * _B_PER_TILE
_ZCHUNKS = ACC_ROWS // 128
_Z_PER_TILE = _ZCHUNKS // 16
_Z_LEFT = _ZCHUNKS - 16 * _Z_PER_TILE
_CP_ROWS = HALF // 16


@functools.partial(
    pl.kernel,
    out_type=(
        jax.ShapeDtypeStruct((NPROT_PAD, OUT), f32),
        jax.ShapeDtypeStruct((NPROT_PAD, 16), f32),
    ),
    mesh=plsc.VectorSubcoreMesh(**_MESH),
    scratch_types=[
        pltpu.VMEM((128,), i32),
        pltpu.VMEM((128,), i32),
        pltpu.VMEM((128, OUT), f32),
        pltpu.VMEM((128, 16), f32),
        pltpu.SemaphoreType.DMA,
        pltpu.VMEM_SHARED((ACC_ROWS, OUT), f32),
        pltpu.VMEM_SHARED((ACC_ROWS, 16), f32),
    ],
)
def _segsum_kernel(z_hbm, src_hbm, dst_hbm, sum_hbm, cnt_hbm,
                   srcbuf, dstbuf, rows, ones, sem, acc, cntacc):
    c = lax.axis_index("c")
    s = lax.axis_index("s")
    base = c * HALF

    zero16 = jnp.zeros((16,), f32)

    def _zero_rows(r, _):
        for g in range(4):
            rows[r, pl.ds(g * 16, 16)] = zero16
        ones[r, :] = zero16
        return 0

    lax.fori_loop(0, 128, _zero_rows, 0)

    def _zero_chunk(q):
        pltpu.sync_copy(rows, acc.at[pl.ds(q * 128, 128), :])
        pltpu.sync_copy(ones, cntacc.at[pl.ds(q * 128, 128), :])

    def _zloop(t, _):
        _zero_chunk(s + 16 * t)
        return 0

    lax.fori_loop(0, _Z_PER_TILE, _zloop, 0)

    @pl.when(s < _Z_LEFT)
    def _():
        _zero_chunk(16 * _Z_PER_TILE + s)

    one16 = jnp.ones((16,), f32)

    def _fill_ones(r, _):
        ones[r, :] = one16
        return 0

    lax.fori_loop(0, 128, _fill_ones, 0)

    plsc.subcore_barrier()

    def _do_batch(bb):
        off = bb * 128
        pltpu.sync_copy(src_hbm.at[pl.ds(off, 128)], srcbuf)
        pltpu.sync_copy(dst_hbm.at[pl.ds(off, 128)], dstbuf)
        for j in range(8):
            sl = pl.ds(j * 16, 16)
            d = dstbuf[sl]
            lo = d - base
            m = (lo >= 0) & (lo < HALF)
            dstbuf[sl] = jnp.where(m, lo, TRASH)
        pltpu.async_copy(z_hbm.at[srcbuf], rows, sem).wait()
        pltpu.sync_copy(rows, acc.at[dstbuf], add=True)
        pltpu.sync_copy(ones, cntacc.at[dstbuf], add=True)

    def _bloop(t, _):
        _do_batch(s + 16 * t)
        return 0

    lax.fori_loop(0, _B_PER_TILE, _bloop, 0)

    @pl.when(s < _B_LEFT)
    def _():
        _do_batch(16 * _B_PER_TILE + s)

    plsc.subcore_barrier()

    lo_r = s * _CP_ROWS
    pltpu.sync_copy(acc.at[pl.ds(lo_r, _CP_ROWS), :],
                    sum_hbm.at[pl.ds(base + lo_r, _CP_ROWS), :])
    pltpu.sync_copy(cntacc.at[pl.ds(lo_r, _CP_ROWS), :],
                    cnt_hbm.at[pl.ds(base + lo_r, _CP_ROWS), :])


_D_BATCHES = L // 128
_D_PER_TILE = _D_BATCHES // 32
_D_LEFT = _D_BATCHES - 32 * _D_PER_TILE


@functools.partial(
    pl.kernel,
    out_type=jax.ShapeDtypeStruct((L, OUT), f32),
    mesh=plsc.VectorSubcoreMesh(**_MESH),
    scratch_types=[
        pltpu.VMEM((128,), i32),
        pltpu.VMEM((128,), i32),
        pltpu.VMEM((128, OUT), f32),
        pltpu.VMEM((128, OUT), f32),
        pltpu.SemaphoreType.DMA,
        pltpu.SemaphoreType.DMA,
    ],
)
def _head_kernel(dp_hbm, pp_hbm, ls_hbm, ld_hbm, out_hbm,
                 lsbuf, ldbuf, dbuf, pbuf, sem1, sem2):
    c = lax.axis_index("c")
    s = lax.axis_index("s")
    w = s * 2 + c

    def _do_batch(bb):
        off = bb * 128
        pltpu.sync_copy(ls_hbm.at[pl.ds(off, 128)], lsbuf)
        pltpu.sync_copy(ld_hbm.at[pl.ds(off, 128)], ldbuf)
        cp1 = pltpu.async_copy(dp_hbm.at[lsbuf], dbuf, sem1)
        cp2 = pltpu.async_copy(pp_hbm.at[ldbuf], pbuf, sem2)
        cp1.wait()
        cp2.wait()

        def _add_row(r, _):
            for g in range(4):
                sl = pl.ds(g * 16, 16)
                plsc.addupdate(dbuf.at[r, sl], pbuf[r, sl])
            return 0

        lax.fori_loop(0, 128, _add_row, 0)
        pltpu.sync_copy(dbuf, out_hbm.at[pl.ds(off, 128), :])

    def _dloop(t, _):
        _do_batch(w + 32 * t)
        return 0

    lax.fori_loop(0, _D_PER_TILE, _dloop, 0)

    @pl.when(w < _D_LEFT)
    def _():
        _do_batch(32 * _D_PER_TILE + w)


@jax.jit
def _impl(x_drug, x_protein, W_l, b_l, W_r, W_lin, b_lin,
          edge_index, edge_label_index):
    src = edge_index[0].astype(i32)
    dst = edge_index[1].astype(i32)
    ls = edge_label_index[0].astype(i32)
    ld = edge_label_index[1].astype(i32)

    z, dp = _proj_drug(x_drug, W_l, W_lin)
    summed, cnt16 = _segsum_kernel(z, src, dst)
    pp = _proj_prot(summed[:N_PROT], cnt16[:N_PROT], x_protein, W_r, W_lin,
                    b_l.reshape(1, D), b_lin.reshape(1, OUT))
    return _head_kernel(dp, pp, ls, ld)


def kernel(x_drug, x_protein, W_l, b_l, W_r, W_lin, b_lin,
           edge_index, edge_label_index):
    return _impl(x_drug, x_protein, W_l, b_l, W_r, W_lin, b_lin,
                 edge_index, edge_label_index)


# speedup vs baseline: 2.9705x; 2.9705x over previous
import functools

import jax
import jax.numpy as jnp
from jax import lax
from jax.experimental import pallas as pl
from jax.experimental.pallas import tpu as pltpu
from jax.experimental.pallas import tpu_sc as plsc

N_DRUG = 10000
N_PROT = 50000
D = 128
OUT = 64
E = 320000
L = 320000

HALF = 25088
ACC_ROWS = 25096
TRASH = 25088
NPROT_PAD = 2 * HALF
CROWS = 1576
CNT_ROWS = 1568

f32 = jnp.float32
i32 = jnp.int32


def _proj_drug_body(x_ref, wl_ref, wlin_ref, z_ref, dp_ref):
    w2 = wlin_ref[D:, :]
    wl2 = jnp.dot(wl_ref[:], w2, preferred_element_type=f32)
    x = x_ref[:]
    z_ref[:] = jnp.dot(x, wl2, preferred_element_type=f32)
    dp_ref[:] = jnp.dot(x, wlin_ref[:D, :], preferred_element_type=f32)


def _proj_drug(x_drug, W_l, W_lin):
    blk = 1000
    grid = N_DRUG // blk
    return pl.pallas_call(
        _proj_drug_body,
        grid=(grid,),
        in_specs=[
            pl.BlockSpec((blk, D), lambda i: (i, 0)),
            pl.BlockSpec((D, D), lambda i: (0, 0)),
            pl.BlockSpec((2 * D, OUT), lambda i: (0, 0)),
        ],
        out_specs=[
            pl.BlockSpec((blk, OUT), lambda i: (i, 0)),
            pl.BlockSpec((blk, OUT), lambda i: (i, 0)),
        ],
        out_shape=[
            jax.ShapeDtypeStruct((N_DRUG, OUT), f32),
            jax.ShapeDtypeStruct((N_DRUG, OUT), f32),
        ],
    )(x_drug, W_l, W_lin)


def _prot_body(sum_ref, cnt_ref, xp_ref, wr_ref, wlin_ref, bl_ref, blin_ref, out_ref):
    blk = sum_ref.shape[0]
    w2 = wlin_ref[D:, :]
    wr2 = jnp.dot(wr_ref[:], w2, preferred_element_type=f32)
    b2 = jnp.dot(bl_ref[:], w2, preferred_element_type=f32) + blin_ref[:]
    inv16 = 1.0 / jnp.maximum(cnt_ref[:], 1.0)
    r_sel = lax.broadcasted_iota(i32, (blk, blk // 16), 0) // 16
    j_sel = lax.broadcasted_iota(i32, (blk, blk // 16), 1)
    sel = (r_sel == j_sel).astype(f32)
    rep = jnp.dot(sel, inv16, preferred_element_type=f32)
    lane = lax.broadcasted_iota(i32, (blk, 16), 1)
    rmod = lax.broadcasted_iota(i32, (blk, 16), 0) & 15
    inv = jnp.sum(jnp.where(lane == rmod, rep, 0.0), axis=1, keepdims=True)
    out_ref[:] = (
        sum_ref[:] * inv
        + jnp.dot(xp_ref[:], wr2, preferred_element_type=f32)
        + b2
    )


def _proj_prot(summed, cnt_pack, x_protein, W_r, W_lin, b_l, b_lin):
    blk = 512
    grid = pl.cdiv(N_PROT, blk)
    return pl.pallas_call(
        _prot_body,
        grid=(grid,),
        in_specs=[
            pl.BlockSpec((blk, OUT), lambda i: (i, 0)),
            pl.BlockSpec((blk // 16, 16), lambda i: (i, 0)),
            pl.BlockSpec((blk, D), lambda i: (i, 0)),
            pl.BlockSpec((D, D), lambda i: (0, 0)),
            pl.BlockSpec((2 * D, OUT), lambda i: (0, 0)),
            pl.BlockSpec((1, D), lambda i: (0, 0)),
            pl.BlockSpec((1, OUT), lambda i: (0, 0)),
        ],
        out_specs=pl.BlockSpec((blk, OUT), lambda i: (i, 0)),
        out_shape=jax.ShapeDtypeStruct((N_PROT, OUT), f32),
    )(summed, cnt_pack, x_protein, W_r, W_lin, b_l, b_lin)


_MESH = dict(core_axis_name="c", subcore_axis_name="s")

_B_BATCHES = E // 128
_B_PER_TILE = _B_BATCHES // 16
_B_LEFT = _B_BATCHES - 16 * _B_PER_TILE
_ZCHUNKS = HALF // 128
_Z_PER_TILE = _ZCHUNKS // 16
_Z_LEFT = _ZCHUNKS - 16 * _Z_PER_TILE
_CP_ROWS = HALF // 16


@functools.partial(
    pl.kernel,
    out_type=(
        jax.ShapeDtypeStruct((NPROT_PAD, OUT), f32),
        jax.ShapeDtypeStruct((2 * CNT_ROWS, 16), f32),
    ),
    mesh=plsc.VectorSubcoreMesh(**_MESH),
    compiler_params=pltpu.CompilerParams(use_tc_tiling_on_sc=False,
                                         needs_layout_passes=False),
    scratch_types=[
        pltpu.VMEM((128,), i32),
        pltpu.VMEM((128,), i32),
        pltpu.VMEM((128,), i32),
        pltpu.VMEM((128, OUT), f32),
        pltpu.VMEM((128, 16), f32),
        pltpu.SemaphoreType.DMA,
        pltpu.VMEM_SHARED((ACC_ROWS, OUT), f32),
        pltpu.VMEM_SHARED((CROWS, 16), f32),
    ],
)
def _segsum_kernel(z_hbm, src_hbm, dst_hbm, sum_hbm, cnt_hbm,
                   srcbuf, dstbuf, rowidx, rows, onehot, sem, acc, cntacc):
    c = lax.axis_index("c")
    s = lax.axis_index("s")
    base = c * HALF

    zero16 = jnp.zeros((16,), f32)
    one16 = jnp.ones((16,), f32)
    lane16 = jnp.arange(16, dtype=i32)

    def _zero_rows(r, _):
        for g in range(4):
            rows[r, pl.ds(g * 16, 16)] = zero16
        onehot[r, :] = zero16
        return 0

    lax.fori_loop(0, 128, _zero_rows, 0)

    def _zloop(t, _):
        q = s + 16 * t
        pltpu.sync_copy(rows, acc.at[pl.ds(q * 128, 128), :])
        return 0

    lax.fori_loop(0, _Z_PER_TILE, _zloop, 0)

    @pl.when(s < _Z_LEFT)
    def _():
        q = 16 * _Z_PER_TILE + s
        pltpu.sync_copy(rows, acc.at[pl.ds(q * 128, 128), :])

    @pl.when(s == 4)
    def _():
        pltpu.sync_copy(rows.at[pl.ds(0, 8), :], acc.at[pl.ds(HALF, 8), :])

    @pl.when(s < 12)
    def _():
        pltpu.sync_copy(onehot, cntacc.at[pl.ds(s * 128, 128), :])

    @pl.when(s == 12)
    def _():
        pltpu.sync_copy(onehot.at[pl.ds(0, 40), :],
                        cntacc.at[pl.ds(1536, 40), :])

    plsc.subcore_barrier()

    def _do_batch(bb):
        off = bb * 128
        pltpu.sync_copy(src_hbm.at[pl.ds(off, 128)], srcbuf)
        pltpu.sync_copy(dst_hbm.at[pl.ds(off, 128)], dstbuf)
        for j in range(8):
            sl = pl.ds(j * 16, 16)
            d = dstbuf[sl]
            lo = d - base
            m = (lo >= 0) & (lo < HALF)
            loc = jnp.where(m, lo, TRASH)
            dstbuf[sl] = loc
            rowidx[sl] = lax.shift_right_logical(loc, 4)
            pos = loc & 15
            plsc.store_scatter(onehot, [lane16 + j * 16, pos], one16)
        pltpu.async_copy(z_hbm.at[srcbuf], rows, sem).wait()
        pltpu.sync_copy(rows, acc.at[dstbuf], add=True)
        pltpu.sync_copy(onehot, cntacc.at[rowidx], add=True)
        for j in range(8):
            sl = pl.ds(j * 16, 16)
            pos = dstbuf[sl] & 15
            plsc.store_scatter(onehot, [lane16 + j * 16, pos], zero16)

    def _bloop(t, _):
        _do_batch(s + 16 * t)
        return 0

    lax.fori_loop(0, _B_PER_TILE, _bloop, 0)

    @pl.when(s < _B_LEFT)
    def _():
        _do_batch(16 * _B_PER_TILE + s)

    plsc.subcore_barrier()

    lo_r = s * _CP_ROWS
    pltpu.sync_copy(acc.at[pl.ds(lo_r, _CP_ROWS), :],
                    sum_hbm.at[pl.ds(base + lo_r, _CP_ROWS), :])

    @pl.when(s == 0)
    def _():
        pltpu.sync_copy(cntacc.at[pl.ds(0, CNT_ROWS), :],
                        cnt_hbm.at[pl.ds(c * CNT_ROWS, CNT_ROWS), :])


_D_BATCHES = L // 128
_D_PER_TILE = _D_BATCHES // 32
_D_LEFT = _D_BATCHES - 32 * _D_PER_TILE


@functools.partial(
    pl.kernel,
    out_type=jax.ShapeDtypeStruct((L, OUT), f32),
    mesh=plsc.VectorSubcoreMesh(**_MESH),
    compiler_params=pltpu.CompilerParams(use_tc_tiling_on_sc=False),
    scratch_types=[
        pltpu.VMEM((128,), i32),
        pltpu.VMEM((128,), i32),
        pltpu.VMEM((128, OUT), f32),
        pltpu.VMEM((128, OUT), f32),
        pltpu.SemaphoreType.DMA,
        pltpu.SemaphoreType.DMA,
    ],
)
def _head_kernel(dp_hbm, pp_hbm, ls_hbm, ld_hbm, out_hbm,
                 lsbuf, ldbuf, dbuf, pbuf, sem1, sem2):
    c = lax.axis_index("c")
    s = lax.axis_index("s")
    w = s * 2 + c

    def _do_batch(bb):
        off = bb * 128
        pltpu.sync_copy(ls_hbm.at[pl.ds(off, 128)], lsbuf)
        pltpu.sync_copy(ld_hbm.at[pl.ds(off, 128)], ldbuf)
        cp1 = pltpu.async_copy(dp_hbm.at[lsbuf], dbuf, sem1)
        cp2 = pltpu.async_copy(pp_hbm.at[ldbuf], pbuf, sem2)
        cp1.wait()
        cp2.wait()

        def _add_row(r, _):
            for g in range(4):
                sl = pl.ds(g * 16, 16)
                plsc.addupdate(dbuf.at[r, sl], pbuf[r, sl])
            return 0

        lax.fori_loop(0, 128, _add_row, 0)
        pltpu.sync_copy(dbuf, out_hbm.at[pl.ds(off, 128), :])

    def _dloop(t, _):
        _do_batch(w + 32 * t)
        return 0

    lax.fori_loop(0, _D_PER_TILE, _dloop, 0)

    @pl.when(w < _D_LEFT)
    def _():
        _do_batch(32 * _D_PER_TILE + w)


@jax.jit
def _impl(x_drug, x_protein, W_l, b_l, W_r, W_lin, b_lin,
          edge_index, edge_label_index):
    src = edge_index[0].astype(i32)
    dst = edge_index[1].astype(i32)
    ls = edge_label_index[0].astype(i32)
    ld = edge_label_index[1].astype(i32)

    z, dp = _proj_drug(x_drug, W_l, W_lin)
    summed, cnt_pack = _segsum_kernel(z, src, dst)
    pp = _proj_prot(summed, cnt_pack, x_protein, W_r, W_lin,
                    b_l.reshape(1, D), b_lin.reshape(1, OUT))
    return _head_kernel(dp, pp, ls, ld)


def kernel(x_drug, x_protein, W_l, b_l, W_r, W_lin, b_lin,
           edge_index, edge_label_index):
    return _impl(x_drug, x_protein, W_l, b_l, W_r, W_lin, b_lin,
                 edge_index, edge_label_index)


# speedup vs baseline: 4.0749x; 1.3718x over previous
import functools

import jax
import jax.numpy as jnp
from jax import lax
from jax.experimental import pallas as pl
from jax.experimental.pallas import tpu as pltpu
from jax.experimental.pallas import tpu_sc as plsc

N_DRUG = 10000
N_PROT = 50000
D = 128
OUT = 64
E = 320000
L = 320000

ACC_ROWS = 50048
CROWS = 3136

f32 = jnp.float32
i32 = jnp.int32


def _proj_drug_body(x_ref, wl_ref, wlin_ref, zlo_ref, zhi_ref, dp_ref):
    w2 = wlin_ref[D:, :]
    wl2 = jnp.dot(wl_ref[:], w2, preferred_element_type=f32)
    x = x_ref[:]
    z = jnp.dot(x, wl2, preferred_element_type=f32)
    zlo_ref[:] = z[:, :32]
    zhi_ref[:] = z[:, 32:]
    dp_ref[:] = jnp.dot(x, wlin_ref[:D, :], preferred_element_type=f32)


def _proj_drug(x_drug, W_l, W_lin):
    blk = 1000
    grid = N_DRUG // blk
    return pl.pallas_call(
        _proj_drug_body,
        grid=(grid,),
        in_specs=[
            pl.BlockSpec((blk, D), lambda i: (i, 0)),
            pl.BlockSpec((D, D), lambda i: (0, 0)),
            pl.BlockSpec((2 * D, OUT), lambda i: (0, 0)),
        ],
        out_specs=[
            pl.BlockSpec((blk, 32), lambda i: (i, 0)),
            pl.BlockSpec((blk, 32), lambda i: (i, 0)),
            pl.BlockSpec((blk, OUT), lambda i: (i, 0)),
        ],
        out_shape=[
            jax.ShapeDtypeStruct((N_DRUG, 32), f32),
            jax.ShapeDtypeStruct((N_DRUG, 32), f32),
            jax.ShapeDtypeStruct((N_DRUG, OUT), f32),
        ],
    )(x_drug, W_l, W_lin)


def _prot_body(sum_ref, c0_ref, c1_ref, xp_ref, wr_ref, wlin_ref, bl_ref,
               blin_ref, out_ref):
    blk = sum_ref.shape[0]
    w2 = wlin_ref[D:, :]
    wr2 = jnp.dot(wr_ref[:], w2, preferred_element_type=f32)
    b2 = jnp.dot(bl_ref[:], w2, preferred_element_type=f32) + blin_ref[:]
    cnt16 = c0_ref[:] + c1_ref[:]
    inv16 = 1.0 / jnp.maximum(cnt16, 1.0)
    r_sel = lax.broadcasted_iota(i32, (blk, blk // 16), 0) // 16
    j_sel = lax.broadcasted_iota(i32, (blk, blk // 16), 1)
    sel = (r_sel == j_sel).astype(f32)
    rep = jnp.dot(sel, inv16, preferred_element_type=f32)
    lane = lax.broadcasted_iota(i32, (blk, 16), 1)
    rmod = lax.broadcasted_iota(i32, (blk, 16), 0) & 15
    inv = jnp.sum(jnp.where(lane == rmod, rep, 0.0), axis=1, keepdims=True)
    out_ref[:] = (
        sum_ref[:] * inv
        + jnp.dot(xp_ref[:], wr2, preferred_element_type=f32)
        + b2
    )


def _proj_prot(summed, cnt_pack, x_protein, W_r, W_lin, b_l, b_lin):
    blk = 512
    grid = pl.cdiv(N_PROT, blk)
    cblk = blk // 16
    return pl.pallas_call(
        _prot_body,
        grid=(grid,),
        in_specs=[
            pl.BlockSpec((blk, OUT), lambda i: (i, 0)),
            pl.BlockSpec((cblk, 16), lambda i: (i, 0)),
            pl.BlockSpec((cblk, 16), lambda i: (i + CROWS // cblk, 0)),
            pl.BlockSpec((blk, D), lambda i: (i, 0)),
            pl.BlockSpec((D, D), lambda i: (0, 0)),
            pl.BlockSpec((2 * D, OUT), lambda i: (0, 0)),
            pl.BlockSpec((1, D), lambda i: (0, 0)),
            pl.BlockSpec((1, OUT), lambda i: (0, 0)),
        ],
        out_specs=pl.BlockSpec((blk, OUT), lambda i: (i, 0)),
        out_shape=jax.ShapeDtypeStruct((N_PROT, OUT), f32),
    )(summed, cnt_pack, cnt_pack, x_protein, W_r, W_lin, b_l, b_lin)


_MESH = dict(core_axis_name="c", subcore_axis_name="s")
_SC_PARAMS = pltpu.CompilerParams(use_tc_tiling_on_sc=False,
                                  needs_layout_passes=False)

_B_BATCHES = E // 128
_B_PER_TILE = _B_BATCHES // 16
_B_LEFT = _B_BATCHES - 16 * _B_PER_TILE
_B_CHALF = _B_BATCHES // 2
_ZCHUNKS = ACC_ROWS // 128
_Z_PER_TILE = _ZCHUNKS // 16
_Z_LEFT = _ZCHUNKS - 16 * _Z_PER_TILE
_CP_ROWS = ACC_ROWS // 16


@functools.partial(
    pl.kernel,
    out_type=(
        jax.ShapeDtypeStruct((ACC_ROWS, OUT), f32),
        jax.ShapeDtypeStruct((2 * CROWS, 16), f32),
    ),
    mesh=plsc.VectorSubcoreMesh(**_MESH),
    compiler_params=_SC_PARAMS,
    scratch_types=[
        pltpu.VMEM((2, 128), i32),
        pltpu.VMEM((128,), i32),
        pltpu.VMEM((128,), i32),
        pltpu.VMEM((128, 32), f32),
        pltpu.VMEM((128, 16), f32),
        pltpu.SemaphoreType.DMA,
        pltpu.VMEM_SHARED((ACC_ROWS, 32), f32),
        pltpu.VMEM_SHARED((CROWS, 16), f32),
    ],
)
def _segsum_kernel(z_hbm, e_hbm, sum_hbm, cnt_hbm,
                   ebuf, gidx, rowidx, rows, onehot, sem, acc, cntacc):
    c = lax.axis_index("c")
    s = lax.axis_index("s")

    zero16 = jnp.zeros((16,), f32)
    one16 = jnp.ones((16,), f32)
    lane16 = jnp.arange(16, dtype=i32)

    def _zero_rows(r, _):
        for g in range(2):
            rows[r, pl.ds(g * 16, 16)] = zero16
        onehot[r, :] = zero16
        return 0

    lax.fori_loop(0, 128, _zero_rows, 0)

    def _zloop(t, _):
        q = s + 16 * t
        pltpu.sync_copy(rows, acc.at[pl.ds(q * 128, 128), :])
        return 0

    lax.fori_loop(0, _Z_PER_TILE, _zloop, 0)

    @pl.when(s < _Z_LEFT)
    def _():
        q = 16 * _Z_PER_TILE + s
        pltpu.sync_copy(rows, acc.at[pl.ds(q * 128, 128), :])

    @pl.when(s < 8)
    def _():
        pltpu.sync_copy(onehot, cntacc.at[pl.ds((16 + s) * 128, 128), :])
        pltpu.sync_copy(onehot, cntacc.at[pl.ds(s * 128, 128), :])

    @pl.when((s >= 8) & (s < 16))
    def _():
        pltpu.sync_copy(onehot, cntacc.at[pl.ds(s * 128, 128), :])

    @pl.when(s == 8)
    def _():
        pltpu.sync_copy(onehot.at[pl.ds(0, 64), :],
                        cntacc.at[pl.ds(3072, 64), :])

    plsc.subcore_barrier()

    def _do_batch(bb):
        off = bb * 128
        pltpu.sync_copy(e_hbm.at[:, pl.ds(off, 128)], ebuf)
        for j in range(8):
            sl = pl.ds(j * 16, 16)
            gidx[sl] = ebuf[0, sl] + c * N_DRUG
        pltpu.async_copy(z_hbm.at[gidx], rows, sem).wait()
        pltpu.sync_copy(rows, acc.at[ebuf.at[1]], add=True)

        in_half = (bb >= c * _B_CHALF) & (bb < (c + 1) * _B_CHALF)

        @pl.when(in_half)
        def _():
            for j in range(8):
                sl = pl.ds(j * 16, 16)
                d = ebuf[1, sl]
                rowidx[sl] = lax.shift_right_logical(d, 4)
                plsc.store_scatter(onehot, [lane16 + j * 16, d & 15], one16)
            pltpu.sync_copy(onehot, cntacc.at[rowidx], add=True)
            for j in range(8):
                sl = pl.ds(j * 16, 16)
                d = ebuf[1, sl]
                plsc.store_scatter(onehot, [lane16 + j * 16, d & 15], zero16)

    def _bloop(t, _):
        _do_batch(s + 16 * t)
        return 0

    lax.fori_loop(0, _B_PER_TILE, _bloop, 0)

    @pl.when(s < _B_LEFT)
    def _():
        _do_batch(16 * _B_PER_TILE + s)

    plsc.subcore_barrier()

    lo_r = s * _CP_ROWS
    pltpu.sync_copy(acc.at[pl.ds(lo_r, _CP_ROWS), :],
                    sum_hbm.at[pl.ds(lo_r, _CP_ROWS), pl.ds(c * 32, 32)])

    @pl.when(s == 0)
    def _():
        pltpu.sync_copy(cntacc, cnt_hbm.at[pl.ds(c * CROWS, CROWS), :])


_D_PER_TILE = L // 32
_D_NB = _D_PER_TILE // 128
_D_TAIL = _D_PER_TILE - _D_NB * 128


@functools.partial(
    pl.kernel,
    out_type=jax.ShapeDtypeStruct((L, OUT), f32),
    mesh=plsc.VectorSubcoreMesh(**_MESH),
    compiler_params=_SC_PARAMS,
    scratch_types=[
        pltpu.VMEM((_D_PER_TILE,), i32),
        pltpu.VMEM((_D_PER_TILE,), i32),
        pltpu.VMEM((2, 128, OUT), f32),
        pltpu.VMEM((2, 128, OUT), f32),
        pltpu.VMEM((2, 128, OUT), f32),
        pltpu.SemaphoreType.DMA,
        pltpu.SemaphoreType.DMA,
        pltpu.SemaphoreType.DMA,
        pltpu.SemaphoreType.DMA,
    ],
)
def _head_kernel(dp_hbm, pp_hbm, ls_hbm, ld_hbm, out_hbm,
                 lsbig, ldbig, dbuf, pbuf, obuf, semg0, semg1, semo0, semo1):
    c = lax.axis_index("c")
    s = lax.axis_index("s")
    w = s * 2 + c
    tbase = w * _D_PER_TILE
    semg = (semg0, semg1)
    semo = (semo0, semo1)

    pltpu.sync_copy(ls_hbm.at[pl.ds(tbase, _D_PER_TILE)], lsbig)
    pltpu.sync_copy(ld_hbm.at[pl.ds(tbase, _D_PER_TILE)], ldbig)

    def _fire(b, p):
        ioff = b * 128
        pltpu.async_copy(dp_hbm.at[lsbig.at[pl.ds(ioff, 128)]],
                         dbuf.at[p], semg[p])
        pltpu.async_copy(pp_hbm.at[ldbig.at[pl.ds(ioff, 128)]],
                         pbuf.at[p], semg[p])

    def _wait_gather(b, p):
        ioff = b * 128
        pltpu.make_async_copy(dp_hbm.at[lsbig.at[pl.ds(ioff, 128)]],
                              dbuf.at[p], semg[p]).wait()
        pltpu.make_async_copy(pp_hbm.at[ldbig.at[pl.ds(ioff, 128)]],
                              pbuf.at[p], semg[p]).wait()

    def _add(p):
        def _row(r, _):
            for g in range(4):
                sl = pl.ds(g * 16, 16)
                obuf[p, r, sl] = dbuf[p, r, sl] + pbuf[p, r, sl]
            return 0

        lax.fori_loop(0, 128, _row, 0)

    def _out_desc(b, p):
        return pltpu.make_async_copy(
            obuf.at[p], out_hbm.at[pl.ds(tbase + b * 128, 128), :], semo[p])

    _fire(0, 0)
    _fire(1, 1)

    def _pair(t2, _):
        for p in range(2):
            b = 2 * t2 + p
            _wait_gather(b, p)

            @pl.when(b >= 2)
            def _():
                _out_desc(b - 2, p).wait()

            _add(p)

            @pl.when(b + 2 < _D_NB)
            def _():
                _fire(b + 2, p)

            pltpu.async_copy(
                obuf.at[p], out_hbm.at[pl.ds(tbase + b * 128, 128), :],
                semo[p])
        return 0

    lax.fori_loop(0, _D_NB // 2, _pair, 0)

    _out_desc(_D_NB - 2, 0).wait()
    _out_desc(_D_NB - 1, 1).wait()

    toff = _D_NB * 128
    cp1 = pltpu.async_copy(dp_hbm.at[lsbig.at[pl.ds(toff, _D_TAIL)]],
                           dbuf.at[0, pl.ds(0, _D_TAIL)], semg0)
    cp2 = pltpu.async_copy(pp_hbm.at[ldbig.at[pl.ds(toff, _D_TAIL)]],
                           pbuf.at[0, pl.ds(0, _D_TAIL)], semg0)
    cp1.wait()
    cp2.wait()

    def _trow(r, _):
        for g in range(4):
            sl = pl.ds(g * 16, 16)
            obuf[0, r, sl] = dbuf[0, r, sl] + pbuf[0, r, sl]
        return 0

    lax.fori_loop(0, _D_TAIL, _trow, 0)
    pltpu.sync_copy(obuf.at[0, pl.ds(0, _D_TAIL)],
                    out_hbm.at[pl.ds(tbase + toff, _D_TAIL), :])


@jax.jit
def _impl(x_drug, x_protein, W_l, b_l, W_r, W_lin, b_lin,
          edge_index, edge_label_index):
    eidx = edge_index.astype(i32)
    ls = edge_label_index[0].astype(i32)
    ld = edge_label_index[1].astype(i32)

    zlo, zhi, dp = _proj_drug(x_drug, W_l, W_lin)
    zsplit = jnp.concatenate([zlo, zhi], axis=0)
    summed, cnt_pack = _segsum_kernel(zsplit, eidx)
    pp = _proj_prot(summed, cnt_pack, x_protein, W_r, W_lin,
                    b_l.reshape(1, D), b_lin.reshape(1, OUT))
    return _head_kernel(dp, pp, ls, ld)


def kernel(x_drug, x_protein, W_l, b_l, W_r, W_lin, b_lin,
           edge_index, edge_label_index):
    return _impl(x_drug, x_protein, W_l, b_l, W_r, W_lin, b_lin,
                 edge_index, edge_label_index)


# speedup vs baseline: 5.2083x; 1.2781x over previous
import functools

import jax
import jax.numpy as jnp
from jax import lax
from jax.experimental import pallas as pl
from jax.experimental.pallas import tpu as pltpu
from jax.experimental.pallas import tpu_sc as plsc

N_DRUG = 10000
N_PROT = 50000
D = 128
OUT = 64
E = 320000
L = 320000

ACC_ROWS = 50048
CROWS = 3136

f32 = jnp.float32
i32 = jnp.int32


def _proj_drug_body(x_ref, wl_ref, wlin_ref, zlo_ref, zhi_ref, dp_ref):
    w2 = wlin_ref[D:, :]
    wl2 = jnp.dot(wl_ref[:], w2, preferred_element_type=f32)
    x = x_ref[:]
    z = jnp.dot(x, wl2, preferred_element_type=f32)
    zlo_ref[:] = z[:, :32]
    zhi_ref[:] = z[:, 32:]
    dp_ref[:] = jnp.dot(x, wlin_ref[:D, :], preferred_element_type=f32)


def _proj_drug(x_drug, W_l, W_lin):
    blk = 1000
    grid = N_DRUG // blk
    return pl.pallas_call(
        _proj_drug_body,
        grid=(grid,),
        in_specs=[
            pl.BlockSpec((blk, D), lambda i: (i, 0)),
            pl.BlockSpec((D, D), lambda i: (0, 0)),
            pl.BlockSpec((2 * D, OUT), lambda i: (0, 0)),
        ],
        out_specs=[
            pl.BlockSpec((blk, 32), lambda i: (i, 0)),
            pl.BlockSpec((blk, 32), lambda i: (i, 0)),
            pl.BlockSpec((blk, OUT), lambda i: (i, 0)),
        ],
        out_shape=[
            jax.ShapeDtypeStruct((N_DRUG, 32), f32),
            jax.ShapeDtypeStruct((N_DRUG, 32), f32),
            jax.ShapeDtypeStruct((N_DRUG, OUT), f32),
        ],
    )(x_drug, W_l, W_lin)


def _prot_body(sum_ref, c0_ref, c1_ref, xp_ref, wr_ref, wlin_ref, bl_ref,
               blin_ref, out_ref):
    blk = sum_ref.shape[0]
    w2 = wlin_ref[D:, :]
    wr2 = jnp.dot(wr_ref[:], w2, preferred_element_type=f32)
    b2 = jnp.dot(bl_ref[:], w2, preferred_element_type=f32) + blin_ref[:]
    cnt16 = c0_ref[:] + c1_ref[:]
    inv16 = 1.0 / jnp.maximum(cnt16, 1.0)
    r_sel = lax.broadcasted_iota(i32, (blk, blk // 16), 0) // 16
    j_sel = lax.broadcasted_iota(i32, (blk, blk // 16), 1)
    sel = (r_sel == j_sel).astype(f32)
    rep = jnp.dot(sel, inv16, preferred_element_type=f32)
    lane = lax.broadcasted_iota(i32, (blk, 16), 1)
    rmod = lax.broadcasted_iota(i32, (blk, 16), 0) & 15
    inv = jnp.sum(jnp.where(lane == rmod, rep, 0.0), axis=1, keepdims=True)
    out_ref[:] = (
        sum_ref[:] * inv
        + jnp.dot(xp_ref[:], wr2, preferred_element_type=f32)
        + b2
    )


def _proj_prot(summed, cnt_pack, x_protein, W_r, W_lin, b_l, b_lin):
    blk = 512
    grid = pl.cdiv(N_PROT, blk)
    cblk = blk // 16
    return pl.pallas_call(
        _prot_body,
        grid=(grid,),
        in_specs=[
            pl.BlockSpec((blk, OUT), lambda i: (i, 0)),
            pl.BlockSpec((cblk, 16), lambda i: (i, 0)),
            pl.BlockSpec((cblk, 16), lambda i: (i + CROWS // cblk, 0)),
            pl.BlockSpec((blk, D), lambda i: (i, 0)),
            pl.BlockSpec((D, D), lambda i: (0, 0)),
            pl.BlockSpec((2 * D, OUT), lambda i: (0, 0)),
            pl.BlockSpec((1, D), lambda i: (0, 0)),
            pl.BlockSpec((1, OUT), lambda i: (0, 0)),
        ],
        out_specs=pl.BlockSpec((blk, OUT), lambda i: (i, 0)),
        out_shape=jax.ShapeDtypeStruct((N_PROT, OUT), f32),
    )(summed, cnt_pack, cnt_pack, x_protein, W_r, W_lin, b_l, b_lin)


_MESH = dict(core_axis_name="c", subcore_axis_name="s")
_SC_PARAMS = pltpu.CompilerParams(use_tc_tiling_on_sc=False,
                                  needs_layout_passes=False)

_B_ETILE = E // 16
_B_NB = _B_ETILE // 128
_B_TAIL = _B_ETILE - _B_NB * 128
_ZCHUNKS = ACC_ROWS // 128
_Z_PER_TILE = _ZCHUNKS // 16
_Z_LEFT = _ZCHUNKS - 16 * _Z_PER_TILE
_CP_ROWS = ACC_ROWS // 16


@functools.partial(
    pl.kernel,
    out_type=(
        jax.ShapeDtypeStruct((ACC_ROWS, OUT), f32),
        jax.ShapeDtypeStruct((2 * CROWS, 16), f32),
    ),
    mesh=plsc.VectorSubcoreMesh(**_MESH),
    compiler_params=_SC_PARAMS,
    scratch_types=[
        pltpu.VMEM((2, 2, 512), i32),
        pltpu.VMEM((2, 128), i32),
        pltpu.VMEM((2, 128), i32),
        pltpu.VMEM((2, 128, 32), f32),
        pltpu.VMEM((2, 128, 16), f32),
        pltpu.VMEM((2, 128), i32),
        pltpu.VMEM((2, 128), i32),
        pltpu.SemaphoreType.DMA,
        pltpu.SemaphoreType.DMA,
        pltpu.SemaphoreType.DMA,
        pltpu.SemaphoreType.DMA,
        pltpu.SemaphoreType.DMA,
        pltpu.SemaphoreType.DMA,
        pltpu.SemaphoreType.DMA,
        pltpu.SemaphoreType.DMA,
        pltpu.VMEM_SHARED((ACC_ROWS, 32), f32),
        pltpu.VMEM_SHARED((CROWS, 16), f32),
    ],
)
def _segsum_kernel(z_hbm, e_hbm, sum_hbm, cnt_hbm,
                   ebuf, gidx, didx, rows, onehot, pos2, rowidx2,
                   seme0, seme1, semg0, semg1, sems0, sems1, semc0, semc1,
                   acc, cntacc):
    c = lax.axis_index("c")
    s = lax.axis_index("s")
    seme = (seme0, seme1)
    semg = (semg0, semg1)
    sems = (sems0, sems1)
    semc = (semc0, semc1)
    tstart = s * _B_ETILE

    zero16 = jnp.zeros((16,), f32)
    one16 = jnp.ones((16,), f32)
    lane16 = jnp.arange(16, dtype=i32)

    def _zero_rows(r, _):
        for g in range(2):
            rows[0, r, pl.ds(g * 16, 16)] = zero16
        onehot[0, r, :] = zero16
        onehot[1, r, :] = zero16
        return 0

    lax.fori_loop(0, 128, _zero_rows, 0)

    def _zloop(t, _):
        q = s + 16 * t
        pltpu.sync_copy(rows.at[0], acc.at[pl.ds(q * 128, 128), :])
        return 0

    lax.fori_loop(0, _Z_PER_TILE, _zloop, 0)

    @pl.when(s < _Z_LEFT)
    def _():
        q = 16 * _Z_PER_TILE + s
        pltpu.sync_copy(rows.at[0], acc.at[pl.ds(q * 128, 128), :])

    @pl.when(s < 8)
    def _():
        pltpu.sync_copy(onehot.at[0], cntacc.at[pl.ds((16 + s) * 128, 128), :])
        pltpu.sync_copy(onehot.at[0], cntacc.at[pl.ds(s * 128, 128), :])

    @pl.when((s >= 8) & (s < 16))
    def _():
        pltpu.sync_copy(onehot.at[0], cntacc.at[pl.ds(s * 128, 128), :])

    @pl.when(s == 8)
    def _():
        pltpu.sync_copy(onehot.at[0, pl.ds(0, 64), :],
                        cntacc.at[pl.ds(3072, 64), :])

    plsc.subcore_barrier()

    def _eload_desc(t, e):
        return pltpu.make_async_copy(
            e_hbm.at[:, pl.ds(tstart + t * 512, 512)], ebuf.at[e], seme[e])

    def _fire_eload(t, e):
        pltpu.async_copy(
            e_hbm.at[:, pl.ds(tstart + t * 512, 512)], ebuf.at[e], seme[e])

    def _compute_idx(e, v, p):
        for j in range(8):
            sl = pl.ds(j * 16, 16)
            esl = pl.ds(v * 128 + j * 16, 16)
            gidx[p, sl] = ebuf[e, 0, esl] + c * N_DRUG
            didx[p, sl] = ebuf[e, 1, esl]

    def _fire_gather(p):
        pltpu.async_copy(z_hbm.at[gidx.at[p]], rows.at[p], semg[p])

    def _wait_gather(p):
        pltpu.make_async_copy(z_hbm.at[gidx.at[p]], rows.at[p],
                              semg[p]).wait()

    def _fire_scatter(p):
        pltpu.async_copy(rows.at[p], acc.at[didx.at[p]], sems[p], add=True)

    def _wait_scatter(p):
        pltpu.make_async_copy(rows.at[p], acc.at[didx.at[p]], sems[p]).wait()

    def _wait_cnt(r):
        pltpu.make_async_copy(onehot.at[r], cntacc.at[rowidx2.at[r]],
                              semc[r]).wait()

    def _count_batch(p, r, first):
        if not first:
            _wait_cnt(r)
            for j in range(8):
                sl = pl.ds(j * 16, 16)
                plsc.store_scatter(onehot.at[r], [lane16 + j * 16,
                                                  pos2[r, sl]], zero16)
        for j in range(8):
            sl = pl.ds(j * 16, 16)
            d = didx[p, sl]
            pos = d & 15
            pos2[r, sl] = pos
            rowidx2[r, sl] = lax.shift_right_logical(d, 4)
            plsc.store_scatter(onehot.at[r], [lane16 + j * 16, pos], one16)
        pltpu.async_copy(onehot.at[r], cntacc.at[rowidx2.at[r]], semc[r],
                         add=True)

    _NQ = _B_NB // 4

    def _visit(t, e, en, v, first):
        p = v % 2
        q = 1 - p
        r = v // 2

        def _advance():
            if not first or v >= 1:
                _wait_scatter(q)
            if v < 3:
                _compute_idx(e, v + 1, q)
            else:
                _eload_desc(t + 1, en).wait()
                _compute_idx(en, 0, q)
                if first:
                    _fire_eload(t + 2, e)
                else:
                    @pl.when(t + 2 < _NQ)
                    def _():
                        _fire_eload(t + 2, e)
            _fire_gather(q)

        if first:
            _advance()
        else:
            @pl.when(4 * t + v + 1 < _B_NB)
            def _():
                _advance()

        _wait_gather(p)
        _fire_scatter(p)

        @pl.when(c == p)
        def _():
            _count_batch(p, r, first)

    _fire_eload(0, 0)
    _fire_eload(1, 1)
    _eload_desc(0, 0).wait()
    _compute_idx(0, 0, 0)
    _fire_gather(0)

    for v in range(4):
        _visit(0, 0, 1, v, True)

    def _dquad(i, _):
        t = 1 + 2 * i
        for dv in range(2):
            e = (1 + dv) % 2
            for v in range(4):
                _visit(t + dv, e, 1 - e, v, False)
        return 0

    lax.fori_loop(0, (_NQ - 1) // 2, _dquad, 0)

    _wait_scatter(0)
    _wait_scatter(1)
    _wait_cnt(0)
    _wait_cnt(1)

    pltpu.sync_copy(e_hbm.at[:, pl.ds(tstart + _B_NB * 128, _B_TAIL)],
                    ebuf.at[0, :, pl.ds(0, _B_TAIL)])
    for j in range(2):
        sl = pl.ds(j * 16, 16)
        gidx[0, sl] = ebuf[0, 0, sl] + c * N_DRUG
        didx[0, sl] = ebuf[0, 1, sl]
    tsl = pl.ds(0, _B_TAIL)
    cp1 = pltpu.async_copy(z_hbm.at[gidx.at[0, tsl]],
                           rows.at[0, tsl], semg0)
    cp1.wait()
    pltpu.sync_copy(rows.at[0, tsl], acc.at[didx.at[0, tsl]], add=True)

    @pl.when(c == 0)
    def _():
        for j in range(2):
            sl = pl.ds(j * 16, 16)
            plsc.store_scatter(onehot.at[0], [lane16 + j * 16,
                                              pos2[0, sl]], zero16)
        for j in range(2):
            sl = pl.ds(j * 16, 16)
            d = didx[0, sl]
            pos = d & 15
            rowidx2[0, sl] = lax.shift_right_logical(d, 4)
            plsc.store_scatter(onehot.at[0], [lane16 + j * 16, pos], one16)
        pltpu.sync_copy(onehot.at[0, tsl],
                        cntacc.at[rowidx2.at[0, tsl]], add=True)

    plsc.subcore_barrier()

    lo_r = s * _CP_ROWS
    pltpu.sync_copy(acc.at[pl.ds(lo_r, _CP_ROWS), :],
                    sum_hbm.at[pl.ds(lo_r, _CP_ROWS), pl.ds(c * 32, 32)])

    @pl.when(s == 0)
    def _():
        pltpu.sync_copy(cntacc, cnt_hbm.at[pl.ds(c * CROWS, CROWS), :])


_D_PER_TILE = L // 32
_D_NB = _D_PER_TILE // 128
_D_TAIL = _D_PER_TILE - _D_NB * 128


@functools.partial(
    pl.kernel,
    out_type=jax.ShapeDtypeStruct((L, OUT), f32),
    mesh=plsc.VectorSubcoreMesh(**_MESH),
    compiler_params=_SC_PARAMS,
    scratch_types=[
        pltpu.VMEM((_D_PER_TILE,), i32),
        pltpu.VMEM((_D_PER_TILE,), i32),
        pltpu.VMEM((2, 128, OUT), f32),
        pltpu.VMEM((2, 128, OUT), f32),
        pltpu.VMEM((2, 128, OUT), f32),
        pltpu.SemaphoreType.DMA,
        pltpu.SemaphoreType.DMA,
        pltpu.SemaphoreType.DMA,
        pltpu.SemaphoreType.DMA,
    ],
)
def _head_kernel(dp_hbm, pp_hbm, ls_hbm, ld_hbm, out_hbm,
                 lsbig, ldbig, dbuf, pbuf, obuf, semg0, semg1, semo0, semo1):
    c = lax.axis_index("c")
    s = lax.axis_index("s")
    w = s * 2 + c
    tbase = w * _D_PER_TILE
    semg = (semg0, semg1)
    semo = (semo0, semo1)

    pltpu.sync_copy(ls_hbm.at[pl.ds(tbase, _D_PER_TILE)], lsbig)
    pltpu.sync_copy(ld_hbm.at[pl.ds(tbase, _D_PER_TILE)], ldbig)

    def _fire(b, p):
        ioff = b * 128
        pltpu.async_copy(dp_hbm.at[lsbig.at[pl.ds(ioff, 128)]],
                         dbuf.at[p], semg[p])
        pltpu.async_copy(pp_hbm.at[ldbig.at[pl.ds(ioff, 128)]],
                         pbuf.at[p], semg[p])

    def _wait_gather(b, p):
        ioff = b * 128
        pltpu.make_async_copy(dp_hbm.at[lsbig.at[pl.ds(ioff, 128)]],
                              dbuf.at[p], semg[p]).wait()
        pltpu.make_async_copy(pp_hbm.at[ldbig.at[pl.ds(ioff, 128)]],
                              pbuf.at[p], semg[p]).wait()

    def _add(p):
        def _row(r, _):
            for g in range(4):
                sl = pl.ds(g * 16, 16)
                obuf[p, r, sl] = dbuf[p, r, sl] + pbuf[p, r, sl]
            return 0

        lax.fori_loop(0, 128, _row, 0)

    def _out_desc(b, p):
        return pltpu.make_async_copy(
            obuf.at[p], out_hbm.at[pl.ds(tbase + b * 128, 128), :], semo[p])

    _fire(0, 0)
    _fire(1, 1)

    def _pair(t2, _):
        for p in range(2):
            b = 2 * t2 + p
            _wait_gather(b, p)

            @pl.when(b >= 2)
            def _():
                _out_desc(b - 2, p).wait()

            _add(p)

            @pl.when(b + 2 < _D_NB)
            def _():
                _fire(b + 2, p)

            pltpu.async_copy(
                obuf.at[p], out_hbm.at[pl.ds(tbase + b * 128, 128), :],
                semo[p])
        return 0

    lax.fori_loop(0, _D_NB // 2, _pair, 0)

    _out_desc(_D_NB - 2, 0).wait()
    _out_desc(_D_NB - 1, 1).wait()

    toff = _D_NB * 128
    cp1 = pltpu.async_copy(dp_hbm.at[lsbig.at[pl.ds(toff, _D_TAIL)]],
                           dbuf.at[0, pl.ds(0, _D_TAIL)], semg0)
    cp2 = pltpu.async_copy(pp_hbm.at[ldbig.at[pl.ds(toff, _D_TAIL)]],
                           pbuf.at[0, pl.ds(0, _D_TAIL)], semg0)
    cp1.wait()
    cp2.wait()

    def _trow(r, _):
        for g in range(4):
            sl = pl.ds(g * 16, 16)
            obuf[0, r, sl] = dbuf[0, r, sl] + pbuf[0, r, sl]
        return 0

    lax.fori_loop(0, _D_TAIL, _trow, 0)
    pltpu.sync_copy(obuf.at[0, pl.ds(0, _D_TAIL)],
                    out_hbm.at[pl.ds(tbase + toff, _D_TAIL), :])


@jax.jit
def _impl(x_drug, x_protein, W_l, b_l, W_r, W_lin, b_lin,
          edge_index, edge_label_index):
    eidx = edge_index.astype(i32)
    ls = edge_label_index[0].astype(i32)
    ld = edge_label_index[1].astype(i32)

    zlo, zhi, dp = _proj_drug(x_drug, W_l, W_lin)
    zsplit = jnp.concatenate([zlo, zhi], axis=0)
    summed, cnt_pack = _segsum_kernel(zsplit, eidx)
    pp = _proj_prot(summed, cnt_pack, x_protein, W_r, W_lin,
                    b_l.reshape(1, D), b_lin.reshape(1, OUT))
    return _head_kernel(dp, pp, ls, ld)


def kernel(x_drug, x_protein, W_l, b_l, W_r, W_lin, b_lin,
           edge_index, edge_label_index):
    return _impl(x_drug, x_protein, W_l, b_l, W_r, W_lin, b_lin,
                 edge_index, edge_label_index)


# speedup vs baseline: 5.4474x; 1.0459x over previous
import functools

import jax
import jax.numpy as jnp
from jax import lax
from jax.experimental import pallas as pl
from jax.experimental.pallas import tpu as pltpu
from jax.experimental.pallas import tpu_sc as plsc

N_DRUG = 10000
N_PROT = 50000
D = 128
OUT = 64
E = 320000
L = 320000

ACC_ROWS = 50048
CROWS = 3136

f32 = jnp.float32
i32 = jnp.int32


def _proj_drug_body(x_ref, wl_ref, wlin_ref, zlo_ref, zhi_ref, dp_ref):
    w2 = wlin_ref[D:, :]
    wl2 = jnp.dot(wl_ref[:], w2, preferred_element_type=f32)
    x = x_ref[:]
    z = jnp.dot(x, wl2, preferred_element_type=f32)
    zlo_ref[:] = z[:, :32]
    zhi_ref[:] = z[:, 32:]
    dp = jnp.dot(x, wlin_ref[:D, :], preferred_element_type=f32)
    dp_ref[:] = jnp.concatenate([dp, jnp.zeros_like(dp)], axis=1)


def _proj_drug(x_drug, W_l, W_lin):
    blk = 1000
    grid = N_DRUG // blk
    return pl.pallas_call(
        _proj_drug_body,
        grid=(grid,),
        in_specs=[
            pl.BlockSpec((blk, D), lambda i: (i, 0)),
            pl.BlockSpec((D, D), lambda i: (0, 0)),
            pl.BlockSpec((2 * D, OUT), lambda i: (0, 0)),
        ],
        out_specs=[
            pl.BlockSpec((blk, 32), lambda i: (i, 0)),
            pl.BlockSpec((blk, 32), lambda i: (i, 0)),
            pl.BlockSpec((blk, 2 * OUT), lambda i: (i, 0)),
        ],
        out_shape=[
            jax.ShapeDtypeStruct((N_DRUG, 32), f32),
            jax.ShapeDtypeStruct((N_DRUG, 32), f32),
            jax.ShapeDtypeStruct((N_DRUG, 2 * OUT), f32),
        ],
    )(x_drug, W_l, W_lin)


def _prot_body(sum_ref, c0_ref, c1_ref, xp_ref, wr_ref, wlin_ref, bl_ref,
               blin_ref, out_ref):
    blk = sum_ref.shape[0]
    w2 = wlin_ref[D:, :]
    wr2 = jnp.dot(wr_ref[:], w2, preferred_element_type=f32)
    b2 = jnp.dot(bl_ref[:], w2, preferred_element_type=f32) + blin_ref[:]
    cnt16 = c0_ref[:] + c1_ref[:]
    inv16 = 1.0 / jnp.maximum(cnt16, 1.0)
    r_sel = lax.broadcasted_iota(i32, (blk, blk // 16), 0) // 16
    j_sel = lax.broadcasted_iota(i32, (blk, blk // 16), 1)
    sel = (r_sel == j_sel).astype(f32)
    rep = jnp.dot(sel, inv16, preferred_element_type=f32)
    lane = lax.broadcasted_iota(i32, (blk, 16), 1)
    rmod = lax.broadcasted_iota(i32, (blk, 16), 0) & 15
    inv = jnp.sum(jnp.where(lane == rmod, rep, 0.0), axis=1, keepdims=True)
    pp = (
        sum_ref[:] * inv
        + jnp.dot(xp_ref[:], wr2, preferred_element_type=f32)
        + b2
    )
    out_ref[:] = jnp.concatenate([pp, jnp.zeros_like(pp)], axis=1)


def _proj_prot(summed, cnt_pack, x_protein, W_r, W_lin, b_l, b_lin):
    blk = 512
    grid = pl.cdiv(N_PROT, blk)
    cblk = blk // 16
    return pl.pallas_call(
        _prot_body,
        grid=(grid,),
        in_specs=[
            pl.BlockSpec((blk, OUT), lambda i: (i, 0)),
            pl.BlockSpec((cblk, 16), lambda i: (i, 0)),
            pl.BlockSpec((cblk, 16), lambda i: (i + CROWS // cblk, 0)),
            pl.BlockSpec((blk, D), lambda i: (i, 0)),
            pl.BlockSpec((D, D), lambda i: (0, 0)),
            pl.BlockSpec((2 * D, OUT), lambda i: (0, 0)),
            pl.BlockSpec((1, D), lambda i: (0, 0)),
            pl.BlockSpec((1, OUT), lambda i: (0, 0)),
        ],
        out_specs=pl.BlockSpec((blk, 2 * OUT), lambda i: (i, 0)),
        out_shape=jax.ShapeDtypeStruct((N_PROT, 2 * OUT), f32),
    )(summed, cnt_pack, cnt_pack, x_protein, W_r, W_lin, b_l, b_lin)


_MESH = dict(core_axis_name="c", subcore_axis_name="s")
_SC_PARAMS = pltpu.CompilerParams(use_tc_tiling_on_sc=False,
                                  needs_layout_passes=False)

_B_ETILE = E // 16
_B_NB = _B_ETILE // 128
_B_TAIL = _B_ETILE - _B_NB * 128
_ZCHUNKS = ACC_ROWS // 128
_Z_PER_TILE = _ZCHUNKS // 16
_Z_LEFT = _ZCHUNKS - 16 * _Z_PER_TILE
_CP_ROWS = ACC_ROWS // 16


@functools.partial(
    pl.kernel,
    out_type=(
        jax.ShapeDtypeStruct((ACC_ROWS, OUT), f32),
        jax.ShapeDtypeStruct((2 * CROWS, 16), f32),
    ),
    mesh=plsc.VectorSubcoreMesh(**_MESH),
    compiler_params=_SC_PARAMS,
    scratch_types=[
        pltpu.VMEM((2, 2, 512), i32),
        pltpu.VMEM((2, 128), i32),
        pltpu.VMEM((2, 128), i32),
        pltpu.VMEM((2, 128, 32), f32),
        pltpu.VMEM((2, 128, 16), f32),
        pltpu.VMEM((2, 128), i32),
        pltpu.VMEM((2, 128), i32),
        pltpu.SemaphoreType.DMA,
        pltpu.SemaphoreType.DMA,
        pltpu.SemaphoreType.DMA,
        pltpu.SemaphoreType.DMA,
        pltpu.SemaphoreType.DMA,
        pltpu.SemaphoreType.DMA,
        pltpu.SemaphoreType.DMA,
        pltpu.SemaphoreType.DMA,
        pltpu.VMEM_SHARED((ACC_ROWS, 32), f32),
        pltpu.VMEM_SHARED((CROWS, 16), f32),
    ],
)
def _segsum_kernel(z_hbm, e_hbm, sum_hbm, cnt_hbm,
                   ebuf, gidx, didx, rows, onehot, pos2, rowidx2,
                   seme0, seme1, semg0, semg1, sems0, sems1, semc0, semc1,
                   acc, cntacc):
    c = lax.axis_index("c")
    s = lax.axis_index("s")
    seme = (seme0, seme1)
    semg = (semg0, semg1)
    sems = (sems0, sems1)
    semc = (semc0, semc1)
    tstart = s * _B_ETILE

    zero16 = jnp.zeros((16,), f32)
    one16 = jnp.ones((16,), f32)
    lane16 = jnp.arange(16, dtype=i32)

    def _zero_rows(r, _):
        for g in range(2):
            rows[0, r, pl.ds(g * 16, 16)] = zero16
        onehot[0, r, :] = zero16
        onehot[1, r, :] = zero16
        return 0

    lax.fori_loop(0, 128, _zero_rows, 0)

    def _zloop(t, _):
        q = s + 16 * t
        pltpu.sync_copy(rows.at[0], acc.at[pl.ds(q * 128, 128), :])
        return 0

    lax.fori_loop(0, _Z_PER_TILE, _zloop, 0)

    @pl.when(s < _Z_LEFT)
    def _():
        q = 16 * _Z_PER_TILE + s
        pltpu.sync_copy(rows.at[0], acc.at[pl.ds(q * 128, 128), :])

    @pl.when(s < 8)
    def _():
        pltpu.sync_copy(onehot.at[0], cntacc.at[pl.ds((16 + s) * 128, 128), :])
        pltpu.sync_copy(onehot.at[0], cntacc.at[pl.ds(s * 128, 128), :])

    @pl.when((s >= 8) & (s < 16))
    def _():
        pltpu.sync_copy(onehot.at[0], cntacc.at[pl.ds(s * 128, 128), :])

    @pl.when(s == 8)
    def _():
        pltpu.sync_copy(onehot.at[0, pl.ds(0, 64), :],
                        cntacc.at[pl.ds(3072, 64), :])

    plsc.subcore_barrier()

    def _eload_desc(t, e):
        return pltpu.make_async_copy(
            e_hbm.at[:, pl.ds(tstart + t * 512, 512)], ebuf.at[e], seme[e])

    def _fire_eload(t, e):
        pltpu.async_copy(
            e_hbm.at[:, pl.ds(tstart + t * 512, 512)], ebuf.at[e], seme[e])

    def _compute_idx(e, v, p):
        for j in range(8):
            sl = pl.ds(j * 16, 16)
            esl = pl.ds(v * 128 + j * 16, 16)
            gidx[p, sl] = ebuf[e, 0, esl] + c * N_DRUG
            didx[p, sl] = ebuf[e, 1, esl]

    def _fire_gather(p):
        pltpu.async_copy(z_hbm.at[gidx.at[p]], rows.at[p], semg[p])

    def _wait_gather(p):
        pltpu.make_async_copy(z_hbm.at[gidx.at[p]], rows.at[p],
                              semg[p]).wait()

    def _fire_scatter(p):
        pltpu.async_copy(rows.at[p], acc.at[didx.at[p]], sems[p], add=True)

    def _wait_scatter(p):
        pltpu.make_async_copy(rows.at[p], acc.at[didx.at[p]], sems[p]).wait()

    def _wait_cnt(r):
        pltpu.make_async_copy(onehot.at[r], cntacc.at[rowidx2.at[r]],
                              semc[r]).wait()

    def _count_batch(p, r, first):
        if not first:
            _wait_cnt(r)
            for j in range(8):
                sl = pl.ds(j * 16, 16)
                plsc.store_scatter(onehot.at[r], [lane16 + j * 16,
                                                  pos2[r, sl]], zero16)
        for j in range(8):
            sl = pl.ds(j * 16, 16)
            d = didx[p, sl]
            pos = d & 15
            pos2[r, sl] = pos
            rowidx2[r, sl] = lax.shift_right_logical(d, 4)
            plsc.store_scatter(onehot.at[r], [lane16 + j * 16, pos], one16)
        pltpu.async_copy(onehot.at[r], cntacc.at[rowidx2.at[r]], semc[r],
                         add=True)

    _NQ = _B_NB // 4

    def _visit(t, e, en, v, first):
        p = v % 2
        q = 1 - p
        r = v // 2

        def _advance():
            if not first or v >= 1:
                _wait_scatter(q)
            if v < 3:
                _compute_idx(e, v + 1, q)
            else:
                _eload_desc(t + 1, en).wait()
                _compute_idx(en, 0, q)
                if first:
                    _fire_eload(t + 2, e)
                else:
                    @pl.when(t + 2 < _NQ)
                    def _():
                        _fire_eload(t + 2, e)
            _fire_gather(q)

        if first:
            _advance()
        else:
            @pl.when(4 * t + v + 1 < _B_NB)
            def _():
                _advance()

        _wait_gather(p)
        _fire_scatter(p)

        @pl.when(c == p)
        def _():
            _count_batch(p, r, first)

    _fire_eload(0, 0)
    _fire_eload(1, 1)
    _eload_desc(0, 0).wait()
    _compute_idx(0, 0, 0)
    _fire_gather(0)

    for v in range(4):
        _visit(0, 0, 1, v, True)

    def _dquad(i, _):
        t = 1 + 2 * i
        for dv in range(2):
            e = (1 + dv) % 2
            for v in range(4):
                _visit(t + dv, e, 1 - e, v, False)
        return 0

    lax.fori_loop(0, (_NQ - 1) // 2, _dquad, 0)

    _wait_scatter(0)
    _wait_scatter(1)
    _wait_cnt(0)
    _wait_cnt(1)

    pltpu.sync_copy(e_hbm.at[:, pl.ds(tstart + _B_NB * 128, _B_TAIL)],
                    ebuf.at[0, :, pl.ds(0, _B_TAIL)])
    for j in range(2):
        sl = pl.ds(j * 16, 16)
        gidx[0, sl] = ebuf[0, 0, sl] + c * N_DRUG
        didx[0, sl] = ebuf[0, 1, sl]
    tsl = pl.ds(0, _B_TAIL)
    cp1 = pltpu.async_copy(z_hbm.at[gidx.at[0, tsl]],
                           rows.at[0, tsl], semg0)
    cp1.wait()
    pltpu.sync_copy(rows.at[0, tsl], acc.at[didx.at[0, tsl]], add=True)

    @pl.when(c == 0)
    def _():
        for j in range(2):
            sl = pl.ds(j * 16, 16)
            plsc.store_scatter(onehot.at[0], [lane16 + j * 16,
                                              pos2[0, sl]], zero16)
        for j in range(2):
            sl = pl.ds(j * 16, 16)
            d = didx[0, sl]
            pos = d & 15
            rowidx2[0, sl] = lax.shift_right_logical(d, 4)
            plsc.store_scatter(onehot.at[0], [lane16 + j * 16, pos], one16)
        pltpu.sync_copy(onehot.at[0, tsl],
                        cntacc.at[rowidx2.at[0, tsl]], add=True)

    plsc.subcore_barrier()

    lo_r = s * _CP_ROWS
    pltpu.sync_copy(acc.at[pl.ds(lo_r, _CP_ROWS), :],
                    sum_hbm.at[pl.ds(lo_r, _CP_ROWS), pl.ds(c * 32, 32)])

    @pl.when(s == 0)
    def _():
        pltpu.sync_copy(cntacc, cnt_hbm.at[pl.ds(c * CROWS, CROWS), :])


_D_PER_TILE = L // 32
_D_NB = _D_PER_TILE // 128
_D_TAIL = _D_PER_TILE - _D_NB * 128


@functools.partial(
    pl.kernel,
    out_type=jax.ShapeDtypeStruct((L, OUT), f32),
    mesh=plsc.VectorSubcoreMesh(**_MESH),
    compiler_params=pltpu.CompilerParams(use_tc_tiling_on_sc=True,
                                         needs_layout_passes=False),
    scratch_types=[
        pltpu.VMEM((_D_PER_TILE,), i32),
        pltpu.VMEM((_D_PER_TILE,), i32),
        pltpu.VMEM((2, 128, 2 * OUT), f32),
        pltpu.VMEM((2, 128, 2 * OUT), f32),
        pltpu.VMEM((2, 128, OUT), f32),
        pltpu.SemaphoreType.DMA,
        pltpu.SemaphoreType.DMA,
        pltpu.SemaphoreType.DMA,
        pltpu.SemaphoreType.DMA,
    ],
)
def _head_kernel(dp_hbm, pp_hbm, ls_hbm, ld_hbm, out_hbm,
                 lsbig, ldbig, dbuf, pbuf, obuf, semg0, semg1, semo0, semo1):
    c = lax.axis_index("c")
    s = lax.axis_index("s")
    w = s * 2 + c
    tbase = w * _D_PER_TILE
    semg = (semg0, semg1)
    semo = (semo0, semo1)

    pltpu.sync_copy(ls_hbm.at[pl.ds(tbase, _D_PER_TILE)], lsbig)
    pltpu.sync_copy(ld_hbm.at[pl.ds(tbase, _D_PER_TILE)], ldbig)

    def _fire(b, p):
        ioff = b * 128
        pltpu.async_copy(dp_hbm.at[lsbig.at[pl.ds(ioff, 128)]],
                         dbuf.at[p], semg[p])
        pltpu.async_copy(pp_hbm.at[ldbig.at[pl.ds(ioff, 128)]],
                         pbuf.at[p], semg[p])

    def _wait_gather(b, p):
        ioff = b * 128
        pltpu.make_async_copy(dp_hbm.at[lsbig.at[pl.ds(ioff, 128)]],
                              dbuf.at[p], semg[p]).wait()
        pltpu.make_async_copy(pp_hbm.at[ldbig.at[pl.ds(ioff, 128)]],
                              pbuf.at[p], semg[p]).wait()

    def _add(p):
        def _row(r, _):
            for g in range(4):
                sl = pl.ds(g * 16, 16)
                obuf[p, r, sl] = dbuf[p, r, sl] + pbuf[p, r, sl]
            return 0

        lax.fori_loop(0, 128, _row, 0)

    def _out_desc(b, p):
        return pltpu.make_async_copy(
            obuf.at[p], out_hbm.at[pl.ds(tbase + b * 128, 128), :], semo[p])

    _fire(0, 0)
    _fire(1, 1)

    def _pair(t2, _):
        for p in range(2):
            b = 2 * t2 + p
            _wait_gather(b, p)

            @pl.when(b >= 2)
            def _():
                _out_desc(b - 2, p).wait()

            _add(p)

            @pl.when(b + 2 < _D_NB)
            def _():
                _fire(b + 2, p)

            pltpu.async_copy(
                obuf.at[p], out_hbm.at[pl.ds(tbase + b * 128, 128), :],
                semo[p])
        return 0

    lax.fori_loop(0, _D_NB // 2, _pair, 0)

    _out_desc(_D_NB - 2, 0).wait()
    _out_desc(_D_NB - 1, 1).wait()

    toff = _D_NB * 128
    cp1 = pltpu.async_copy(dp_hbm.at[lsbig.at[pl.ds(toff, _D_TAIL)]],
                           dbuf.at[0, pl.ds(0, _D_TAIL)], semg0)
    cp2 = pltpu.async_copy(pp_hbm.at[ldbig.at[pl.ds(toff, _D_TAIL)]],
                           pbuf.at[0, pl.ds(0, _D_TAIL)], semg0)
    cp1.wait()
    cp2.wait()

    def _trow(r, _):
        for g in range(4):
            sl = pl.ds(g * 16, 16)
            obuf[0, r, sl] = dbuf[0, r, sl] + pbuf[0, r, sl]
        return 0

    lax.fori_loop(0, _D_TAIL, _trow, 0)
    pltpu.sync_copy(obuf.at[0, pl.ds(0, _D_TAIL)],
                    out_hbm.at[pl.ds(tbase + toff, _D_TAIL), :])


@jax.jit
def _impl(x_drug, x_protein, W_l, b_l, W_r, W_lin, b_lin,
          edge_index, edge_label_index):
    eidx = edge_index.astype(i32)
    ls = edge_label_index[0].astype(i32)
    ld = edge_label_index[1].astype(i32)

    zlo, zhi, dp128 = _proj_drug(x_drug, W_l, W_lin)
    zsplit = jnp.concatenate([zlo, zhi], axis=0)
    summed, cnt_pack = _segsum_kernel(zsplit, eidx)
    pp128 = _proj_prot(summed, cnt_pack, x_protein, W_r, W_lin,
                       b_l.reshape(1, D), b_lin.reshape(1, OUT))
    return _head_kernel(dp128, pp128, ls, ld)


def kernel(x_drug, x_protein, W_l, b_l, W_r, W_lin, b_lin,
           edge_index, edge_label_index):
    return _impl(x_drug, x_protein, W_l, b_l, W_r, W_lin, b_lin,
                 edge_index, edge_label_index)


# speedup vs baseline: 5.4485x; 1.0002x over previous
import functools

import jax
import jax.numpy as jnp
from jax import lax
from jax.experimental import layout as jax_layout
from jax.experimental import pallas as pl
from jax.experimental.pallas import tpu as pltpu
from jax.experimental.pallas import tpu_sc as plsc

N_DRUG = 10000
N_PROT = 50000
D = 128
OUT = 64
E = 320000
L = 320000

ACC_ROWS = 50048
CROWS = 3136

f32 = jnp.float32
i32 = jnp.int32


def _proj_drug_body(x_ref, wl_ref, wlin_ref, zlo_ref, zhi_ref, dp_ref):
    w2 = wlin_ref[D:, :]
    wl2 = jnp.dot(wl_ref[:], w2, preferred_element_type=f32)
    x = x_ref[:]
    z = jnp.dot(x, wl2, preferred_element_type=f32)
    zlo_ref[:] = z[:, :32]
    zhi_ref[:] = z[:, 32:]
    dp = jnp.dot(x, wlin_ref[:D, :], preferred_element_type=f32)
    dp_ref[:] = jnp.concatenate([dp, jnp.zeros_like(dp)], axis=1)


def _proj_drug(x_drug, W_l, W_lin):
    blk = 1000
    grid = N_DRUG // blk
    return pl.pallas_call(
        _proj_drug_body,
        grid=(grid,),
        in_specs=[
            pl.BlockSpec((blk, D), lambda i: (i, 0)),
            pl.BlockSpec((D, D), lambda i: (0, 0)),
            pl.BlockSpec((2 * D, OUT), lambda i: (0, 0)),
        ],
        out_specs=[
            pl.BlockSpec((blk, 32), lambda i: (i, 0)),
            pl.BlockSpec((blk, 32), lambda i: (i, 0)),
            pl.BlockSpec((blk, 2 * OUT), lambda i: (i, 0)),
        ],
        out_shape=[
            jax.ShapeDtypeStruct((N_DRUG, 32), f32),
            jax.ShapeDtypeStruct((N_DRUG, 32), f32),
            jax.ShapeDtypeStruct((N_DRUG, 2 * OUT), f32),
        ],
    )(x_drug, W_l, W_lin)


def _prot_body(sum_ref, c0_ref, c1_ref, xp_ref, wr_ref, wlin_ref, bl_ref,
               blin_ref, out_ref):
    blk = sum_ref.shape[0]
    w2 = wlin_ref[D:, :]
    wr2 = jnp.dot(wr_ref[:], w2, preferred_element_type=f32)
    b2 = jnp.dot(bl_ref[:], w2, preferred_element_type=f32) + blin_ref[:]
    cnt16 = c0_ref[:] + c1_ref[:]
    inv16 = 1.0 / jnp.maximum(cnt16, 1.0)
    r_sel = lax.broadcasted_iota(i32, (blk, blk // 16), 0) // 16
    j_sel = lax.broadcasted_iota(i32, (blk, blk // 16), 1)
    sel = (r_sel == j_sel).astype(f32)
    rep = jnp.dot(sel, inv16, preferred_element_type=f32)
    lane = lax.broadcasted_iota(i32, (blk, 16), 1)
    rmod = lax.broadcasted_iota(i32, (blk, 16), 0) & 15
    inv = jnp.sum(jnp.where(lane == rmod, rep, 0.0), axis=1, keepdims=True)
    pp = (
        sum_ref[:] * inv
        + jnp.dot(xp_ref[:], wr2, preferred_element_type=f32)
        + b2
    )
    out_ref[:] = jnp.concatenate([pp, jnp.zeros_like(pp)], axis=1)


def _proj_prot(summed, cnt_pack, x_protein, W_r, W_lin, b_l, b_lin):
    blk = 512
    grid = pl.cdiv(N_PROT, blk)
    cblk = blk // 16
    return pl.pallas_call(
        _prot_body,
        grid=(grid,),
        in_specs=[
            pl.BlockSpec((blk, OUT), lambda i: (i, 0)),
            pl.BlockSpec((cblk, 16), lambda i: (i, 0)),
            pl.BlockSpec((cblk, 16), lambda i: (i + CROWS // cblk, 0)),
            pl.BlockSpec((blk, D), lambda i: (i, 0)),
            pl.BlockSpec((D, D), lambda i: (0, 0)),
            pl.BlockSpec((2 * D, OUT), lambda i: (0, 0)),
            pl.BlockSpec((1, D), lambda i: (0, 0)),
            pl.BlockSpec((1, OUT), lambda i: (0, 0)),
        ],
        out_specs=pl.BlockSpec((blk, 2 * OUT), lambda i: (i, 0)),
        out_shape=jax.ShapeDtypeStruct((N_PROT, 2 * OUT), f32),
    )(summed, cnt_pack, cnt_pack, x_protein, W_r, W_lin, b_l, b_lin)


_MESH = dict(core_axis_name="c", subcore_axis_name="s")
_SC_PARAMS = pltpu.CompilerParams(use_tc_tiling_on_sc=False,
                                  needs_layout_passes=False)

_B_ETILE = E // 16
_B_NB = _B_ETILE // 128
_B_TAIL = _B_ETILE - _B_NB * 128
_ZCHUNKS = ACC_ROWS // 128
_Z_PER_TILE = _ZCHUNKS // 16
_Z_LEFT = _ZCHUNKS - 16 * _Z_PER_TILE
_CP_ROWS = ACC_ROWS // 16


@functools.partial(
    pl.kernel,
    out_type=(
        jax.ShapeDtypeStruct((ACC_ROWS, OUT), f32),
        jax.ShapeDtypeStruct((2 * CROWS, 16), f32),
    ),
    mesh=plsc.VectorSubcoreMesh(**_MESH),
    compiler_params=_SC_PARAMS,
    scratch_types=[
        pltpu.VMEM((2, 2, 512), i32),
        pltpu.VMEM((2, 128), i32),
        pltpu.VMEM((2, 128), i32),
        pltpu.VMEM((2, 128, 32), f32),
        pltpu.VMEM((2, 128, 16), f32),
        pltpu.VMEM((2, 128), i32),
        pltpu.VMEM((2, 128), i32),
        pltpu.SemaphoreType.DMA,
        pltpu.SemaphoreType.DMA,
        pltpu.SemaphoreType.DMA,
        pltpu.SemaphoreType.DMA,
        pltpu.SemaphoreType.DMA,
        pltpu.SemaphoreType.DMA,
        pltpu.SemaphoreType.DMA,
        pltpu.SemaphoreType.DMA,
        pltpu.VMEM_SHARED((ACC_ROWS, 32), f32),
        pltpu.VMEM_SHARED((CROWS, 16), f32),
    ],
)
def _segsum_kernel(z_hbm, e_hbm, sum_hbm, cnt_hbm,
                   ebuf, gidx, didx, rows, onehot, pos2, rowidx2,
                   seme0, seme1, semg0, semg1, sems0, sems1, semc0, semc1,
                   acc, cntacc):
    c = lax.axis_index("c")
    s = lax.axis_index("s")
    seme = (seme0, seme1)
    semg = (semg0, semg1)
    sems = (sems0, sems1)
    semc = (semc0, semc1)
    tstart = s * _B_ETILE

    zero16 = jnp.zeros((16,), f32)
    one16 = jnp.ones((16,), f32)
    lane16 = jnp.arange(16, dtype=i32)

    def _zero_rows(r, _):
        for g in range(2):
            rows[0, r, pl.ds(g * 16, 16)] = zero16
        onehot[0, r, :] = zero16
        onehot[1, r, :] = zero16
        return 0

    lax.fori_loop(0, 128, _zero_rows, 0)

    def _zloop(t, _):
        q = s + 16 * t
        pltpu.sync_copy(rows.at[0], acc.at[pl.ds(q * 128, 128), :])
        return 0

    lax.fori_loop(0, _Z_PER_TILE, _zloop, 0)

    @pl.when(s < _Z_LEFT)
    def _():
        q = 16 * _Z_PER_TILE + s
        pltpu.sync_copy(rows.at[0], acc.at[pl.ds(q * 128, 128), :])

    @pl.when(s < 8)
    def _():
        pltpu.sync_copy(onehot.at[0], cntacc.at[pl.ds((16 + s) * 128, 128), :])
        pltpu.sync_copy(onehot.at[0], cntacc.at[pl.ds(s * 128, 128), :])

    @pl.when((s >= 8) & (s < 16))
    def _():
        pltpu.sync_copy(onehot.at[0], cntacc.at[pl.ds(s * 128, 128), :])

    @pl.when(s == 8)
    def _():
        pltpu.sync_copy(onehot.at[0, pl.ds(0, 64), :],
                        cntacc.at[pl.ds(3072, 64), :])

    plsc.subcore_barrier()

    def _eload_desc(t, e):
        return pltpu.make_async_copy(
            e_hbm.at[:, pl.ds(tstart + t * 512, 512)], ebuf.at[e], seme[e])

    def _fire_eload(t, e):
        pltpu.async_copy(
            e_hbm.at[:, pl.ds(tstart + t * 512, 512)], ebuf.at[e], seme[e])

    def _compute_idx(e, v, p):
        for j in range(8):
            sl = pl.ds(j * 16, 16)
            esl = pl.ds(v * 128 + j * 16, 16)
            gidx[p, sl] = ebuf[e, 0, esl] + c * N_DRUG
            didx[p, sl] = ebuf[e, 1, esl]

    def _fire_gather(p):
        pltpu.async_copy(z_hbm.at[gidx.at[p]], rows.at[p], semg[p])

    def _wait_gather(p):
        pltpu.make_async_copy(z_hbm.at[gidx.at[p]], rows.at[p],
                              semg[p]).wait()

    def _fire_scatter(p):
        pltpu.async_copy(rows.at[p], acc.at[didx.at[p]], sems[p], add=True)

    def _wait_scatter(p):
        pltpu.make_async_copy(rows.at[p], acc.at[didx.at[p]], sems[p]).wait()

    def _wait_cnt(r):
        pltpu.make_async_copy(onehot.at[r], cntacc.at[rowidx2.at[r]],
                              semc[r]).wait()

    def _count_batch(p, r, first):
        if not first:
            _wait_cnt(r)
            for j in range(8):
                sl = pl.ds(j * 16, 16)
                plsc.store_scatter(onehot.at[r], [lane16 + j * 16,
                                                  pos2[r, sl]], zero16)
        for j in range(8):
            sl = pl.ds(j * 16, 16)
            d = didx[p, sl]
            pos = d & 15
            pos2[r, sl] = pos
            rowidx2[r, sl] = lax.shift_right_logical(d, 4)
            plsc.store_scatter(onehot.at[r], [lane16 + j * 16, pos], one16)
        pltpu.async_copy(onehot.at[r], cntacc.at[rowidx2.at[r]], semc[r],
                         add=True)

    _NQ = _B_NB // 4

    def _visit(t, e, en, v, first):
        p = v % 2
        q = 1 - p
        r = v // 2

        def _advance():
            if not first or v >= 1:
                _wait_scatter(q)
            if v < 3:
                _compute_idx(e, v + 1, q)
            else:
                _eload_desc(t + 1, en).wait()
                _compute_idx(en, 0, q)
                if first:
                    _fire_eload(t + 2, e)
                else:
                    @pl.when(t + 2 < _NQ)
                    def _():
                        _fire_eload(t + 2, e)
            _fire_gather(q)

        if first:
            _advance()
        else:
            @pl.when(4 * t + v + 1 < _B_NB)
            def _():
                _advance()

        _wait_gather(p)
        _fire_scatter(p)

        @pl.when(c == p)
        def _():
            _count_batch(p, r, first)

    _fire_eload(0, 0)
    _fire_eload(1, 1)
    _eload_desc(0, 0).wait()
    _compute_idx(0, 0, 0)
    _fire_gather(0)

    for v in range(4):
        _visit(0, 0, 1, v, True)

    def _dquad(i, _):
        t = 1 + 2 * i
        for dv in range(2):
            e = (1 + dv) % 2
            for v in range(4):
                _visit(t + dv, e, 1 - e, v, False)
        return 0

    lax.fori_loop(0, (_NQ - 1) // 2, _dquad, 0)

    _wait_scatter(0)
    _wait_scatter(1)
    _wait_cnt(0)
    _wait_cnt(1)

    pltpu.sync_copy(e_hbm.at[:, pl.ds(tstart + _B_NB * 128, _B_TAIL)],
                    ebuf.at[0, :, pl.ds(0, _B_TAIL)])
    for j in range(2):
        sl = pl.ds(j * 16, 16)
        gidx[0, sl] = ebuf[0, 0, sl] + c * N_DRUG
        didx[0, sl] = ebuf[0, 1, sl]
    tsl = pl.ds(0, _B_TAIL)
    cp1 = pltpu.async_copy(z_hbm.at[gidx.at[0, tsl]],
                           rows.at[0, tsl], semg0)
    cp1.wait()
    pltpu.sync_copy(rows.at[0, tsl], acc.at[didx.at[0, tsl]], add=True)

    @pl.when(c == 0)
    def _():
        for j in range(2):
            sl = pl.ds(j * 16, 16)
            plsc.store_scatter(onehot.at[0], [lane16 + j * 16,
                                              pos2[0, sl]], zero16)
        for j in range(2):
            sl = pl.ds(j * 16, 16)
            d = didx[0, sl]
            pos = d & 15
            rowidx2[0, sl] = lax.shift_right_logical(d, 4)
            plsc.store_scatter(onehot.at[0], [lane16 + j * 16, pos], one16)
        pltpu.sync_copy(onehot.at[0, tsl],
                        cntacc.at[rowidx2.at[0, tsl]], add=True)

    plsc.subcore_barrier()

    lo_r = s * _CP_ROWS
    pltpu.sync_copy(acc.at[pl.ds(lo_r, _CP_ROWS), :],
                    sum_hbm.at[pl.ds(lo_r, _CP_ROWS), pl.ds(c * 32, 32)])

    @pl.when(s == 0)
    def _():
        pltpu.sync_copy(cntacc, cnt_hbm.at[pl.ds(c * CROWS, CROWS), :])


_D_PER_TILE = L // 32
_D_NB = _D_PER_TILE // 128
_D_TAIL = _D_PER_TILE - _D_NB * 128


@functools.partial(
    pl.kernel,
    out_type=jax.ShapeDtypeStruct((L, OUT), f32),
    mesh=plsc.VectorSubcoreMesh(**_MESH),
    compiler_params=pltpu.CompilerParams(use_tc_tiling_on_sc=True,
                                         needs_layout_passes=False),
    scratch_types=[
        pltpu.VMEM((_D_PER_TILE,), i32),
        pltpu.VMEM((_D_PER_TILE,), i32),
        pltpu.VMEM((2, 128, 2 * OUT), f32),
        pltpu.VMEM((2, 128, 2 * OUT), f32),
        pltpu.VMEM((2, 128, OUT), f32),
        pltpu.SemaphoreType.DMA,
        pltpu.SemaphoreType.DMA,
        pltpu.SemaphoreType.DMA,
        pltpu.SemaphoreType.DMA,
    ],
)
def _head_kernel(dp_hbm, pp_hbm, ls_hbm, ld_hbm, out_hbm,
                 lsbig, ldbig, dbuf, pbuf, obuf, semg0, semg1, semo0, semo1):
    c = lax.axis_index("c")
    s = lax.axis_index("s")
    w = s * 2 + c
    tbase = w * _D_PER_TILE
    semg = (semg0, semg1)
    semo = (semo0, semo1)

    pltpu.sync_copy(ls_hbm.at[pl.ds(tbase, _D_PER_TILE)], lsbig)
    pltpu.sync_copy(ld_hbm.at[pl.ds(tbase, _D_PER_TILE)], ldbig)

    def _fire(b, p):
        ioff = b * 128
        pltpu.async_copy(dp_hbm.at[lsbig.at[pl.ds(ioff, 128)]],
                         dbuf.at[p], semg[p])
        pltpu.async_copy(pp_hbm.at[ldbig.at[pl.ds(ioff, 128)]],
                         pbuf.at[p], semg[p])

    def _wait_gather(b, p):
        ioff = b * 128
        pltpu.make_async_copy(dp_hbm.at[lsbig.at[pl.ds(ioff, 128)]],
                              dbuf.at[p], semg[p]).wait()
        pltpu.make_async_copy(pp_hbm.at[ldbig.at[pl.ds(ioff, 128)]],
                              pbuf.at[p], semg[p]).wait()

    def _add(p):
        def _row(r, _):
            for g in range(4):
                sl = pl.ds(g * 16, 16)
                obuf[p, r, sl] = dbuf[p, r, sl] + pbuf[p, r, sl]
            return 0

        lax.fori_loop(0, 128, _row, 0)

    def _out_desc(b, p):
        return pltpu.make_async_copy(
            obuf.at[p], out_hbm.at[pl.ds(tbase + b * 128, 128), :], semo[p])

    _fire(0, 0)
    _fire(1, 1)

    def _pair(t2, _):
        for p in range(2):
            b = 2 * t2 + p
            _wait_gather(b, p)

            @pl.when(b >= 2)
            def _():
                _out_desc(b - 2, p).wait()

            _add(p)

            @pl.when(b + 2 < _D_NB)
            def _():
                _fire(b + 2, p)

            pltpu.async_copy(
                obuf.at[p], out_hbm.at[pl.ds(tbase + b * 128, 128), :],
                semo[p])
        return 0

    lax.fori_loop(0, _D_NB // 2, _pair, 0)

    _out_desc(_D_NB - 2, 0).wait()
    _out_desc(_D_NB - 1, 1).wait()

    toff = _D_NB * 128
    cp1 = pltpu.async_copy(dp_hbm.at[lsbig.at[pl.ds(toff, _D_TAIL)]],
                           dbuf.at[0, pl.ds(0, _D_TAIL)], semg0)
    cp2 = pltpu.async_copy(pp_hbm.at[ldbig.at[pl.ds(toff, _D_TAIL)]],
                           pbuf.at[0, pl.ds(0, _D_TAIL)], semg0)
    cp1.wait()
    cp2.wait()

    def _trow(r, _):
        for g in range(4):
            sl = pl.ds(g * 16, 16)
            obuf[0, r, sl] = dbuf[0, r, sl] + pbuf[0, r, sl]
        return 0

    lax.fori_loop(0, _D_TAIL, _trow, 0)
    pltpu.sync_copy(obuf.at[0, pl.ds(0, _D_TAIL)],
                    out_hbm.at[pl.ds(tbase + toff, _D_TAIL), :])


def _impl(x_drug, x_protein, W_l, b_l, W_r, W_lin, b_lin,
          edge_index, edge_label_index):
    eidx = edge_index.astype(i32)
    ls = edge_label_index[0].astype(i32)
    ld = edge_label_index[1].astype(i32)

    zlo, zhi, dp128 = _proj_drug(x_drug, W_l, W_lin)
    zsplit = jnp.concatenate([zlo, zhi], axis=0)
    summed, cnt_pack = _segsum_kernel(zsplit, eidx)
    pp128 = _proj_prot(summed, cnt_pack, x_protein, W_r, W_lin,
                       b_l.reshape(1, D), b_lin.reshape(1, OUT))
    out = _head_kernel(dp128, pp128, ls, ld)
    try:
        fmt = jax_layout.Format(
            jax_layout.Layout((1, 0)),
            jax.sharding.SingleDeviceSharding(jax.devices()[0]),
        )
        return jax_layout.with_layout_constraint(out, fmt)
    except ValueError:
        return out


_impl_jitted = jax.jit(_impl)


def kernel(x_drug, x_protein, W_l, b_l, W_r, W_lin, b_lin,
           edge_index, edge_label_index):
    return _impl_jitted(x_drug, x_protein, W_l, b_l, W_r, W_lin, b_lin,
                        edge_index, edge_label_index)


# speedup vs baseline: 5.7867x; 1.0621x over previous
import functools

import jax
import jax.numpy as jnp
from jax import lax
from jax.experimental import pallas as pl
from jax.experimental.pallas import tpu as pltpu
from jax.experimental.pallas import tpu_sc as plsc

N_DRUG = 10000
N_PROT = 50000
D = 128
OUT = 64
E = 320000
L = 320000

ACC_ROWS = 50048
CROWS = 3136

f32 = jnp.float32
i32 = jnp.int32


def _proj_drug_body(x_ref, wl_ref, wlin_ref, z_ref, dp_ref):
    half = pl.program_id(0) // 10
    x = x_ref[:]
    w2 = wlin_ref[D:, :]
    wl2 = jnp.dot(wl_ref[:], w2, preferred_element_type=f32)
    z = jnp.dot(x, wl2, preferred_element_type=f32)
    rsel = lax.broadcasted_iota(i32, (OUT, 32), 0)
    csel = lax.broadcasted_iota(i32, (OUT, 32), 1)
    sel = (rsel == csel + 32 * half).astype(f32)
    z_ref[:] = jnp.dot(z, sel, preferred_element_type=f32)
    dp = jnp.dot(x, wlin_ref[:D, :], preferred_element_type=f32)
    dp_ref[:] = jnp.concatenate([dp, jnp.zeros_like(dp)], axis=1)


def _proj_drug(x_drug, W_l, W_lin):
    blk = 1000
    grid = 2 * (N_DRUG // blk)
    return pl.pallas_call(
        _proj_drug_body,
        grid=(grid,),
        in_specs=[
            pl.BlockSpec((blk, D), lambda i: (i % 10, 0)),
            pl.BlockSpec((D, D), lambda i: (0, 0)),
            pl.BlockSpec((2 * D, OUT), lambda i: (0, 0)),
        ],
        out_specs=[
            pl.BlockSpec((blk, 32), lambda i: (i, 0)),
            pl.BlockSpec((blk, 2 * OUT), lambda i: (i % 10, 0)),
        ],
        out_shape=[
            jax.ShapeDtypeStruct((2 * N_DRUG, 32), f32),
            jax.ShapeDtypeStruct((N_DRUG, 2 * OUT), f32),
        ],
    )(x_drug, W_l, W_lin)


def _prot_body(sum_ref, c0_ref, c1_ref, xp_ref, wr_ref, wlin_ref, bl_ref,
               blin_ref, out_ref):
    blk = sum_ref.shape[0]
    w2 = wlin_ref[D:, :]
    wr2 = jnp.dot(wr_ref[:], w2, preferred_element_type=f32)
    b2 = jnp.dot(bl_ref[:], w2, preferred_element_type=f32) + blin_ref[:]
    cnt16 = c0_ref[:] + c1_ref[:]
    inv16 = 1.0 / jnp.maximum(cnt16, 1.0)
    r_sel = lax.broadcasted_iota(i32, (blk, blk // 16), 0) // 16
    j_sel = lax.broadcasted_iota(i32, (blk, blk // 16), 1)
    sel = (r_sel == j_sel).astype(f32)
    rep = jnp.dot(sel, inv16, preferred_element_type=f32)
    lane = lax.broadcasted_iota(i32, (blk, 16), 1)
    rmod = lax.broadcasted_iota(i32, (blk, 16), 0) & 15
    inv = jnp.sum(jnp.where(lane == rmod, rep, 0.0), axis=1, keepdims=True)
    pp = (
        sum_ref[:] * inv
        + jnp.dot(xp_ref[:], wr2, preferred_element_type=f32)
        + b2
    )
    out_ref[:] = jnp.concatenate([pp, jnp.zeros_like(pp)], axis=1)


def _proj_prot(summed, cnt_pack, x_protein, W_r, W_lin, b_l, b_lin):
    blk = 1024
    grid = pl.cdiv(N_PROT, blk)
    cblk = blk // 16
    return pl.pallas_call(
        _prot_body,
        grid=(grid,),
        in_specs=[
            pl.BlockSpec((blk, OUT), lambda i: (i, 0)),
            pl.BlockSpec((cblk, 16), lambda i: (i, 0)),
            pl.BlockSpec((cblk, 16), lambda i: (i + CROWS // cblk, 0)),
            pl.BlockSpec((blk, D), lambda i: (i, 0)),
            pl.BlockSpec((D, D), lambda i: (0, 0)),
            pl.BlockSpec((2 * D, OUT), lambda i: (0, 0)),
            pl.BlockSpec((1, D), lambda i: (0, 0)),
            pl.BlockSpec((1, OUT), lambda i: (0, 0)),
        ],
        out_specs=pl.BlockSpec((blk, 2 * OUT), lambda i: (i, 0)),
        out_shape=jax.ShapeDtypeStruct((N_PROT, 2 * OUT), f32),
    )(summed, cnt_pack, cnt_pack, x_protein, W_r, W_lin, b_l, b_lin)


_MESH = dict(core_axis_name="c", subcore_axis_name="s")
_SC_PARAMS = pltpu.CompilerParams(use_tc_tiling_on_sc=False,
                                  needs_layout_passes=False)

_B_ETILE = E // 16
_B_NB = _B_ETILE // 128
_B_TAIL = _B_ETILE - _B_NB * 128
_ZCHUNKS = ACC_ROWS // 128
_Z_PER_TILE = _ZCHUNKS // 16
_Z_LEFT = _ZCHUNKS - 16 * _Z_PER_TILE
_CP_ROWS = ACC_ROWS // 16


@functools.partial(
    pl.kernel,
    out_type=(
        jax.ShapeDtypeStruct((ACC_ROWS, OUT), f32),
        jax.ShapeDtypeStruct((2 * CROWS, 16), f32),
    ),
    mesh=plsc.VectorSubcoreMesh(**_MESH),
    compiler_params=_SC_PARAMS,
    scratch_types=[
        pltpu.VMEM((2, 2, 512), i32),
        pltpu.VMEM((2, 128), i32),
        pltpu.VMEM((2, 128), i32),
        pltpu.VMEM((2, 128, 32), f32),
        pltpu.VMEM((2, 128, 16), f32),
        pltpu.VMEM((2, 128), i32),
        pltpu.VMEM((2, 128), i32),
        pltpu.SemaphoreType.DMA,
        pltpu.SemaphoreType.DMA,
        pltpu.SemaphoreType.DMA,
        pltpu.SemaphoreType.DMA,
        pltpu.SemaphoreType.DMA,
        pltpu.SemaphoreType.DMA,
        pltpu.SemaphoreType.DMA,
        pltpu.SemaphoreType.DMA,
        pltpu.VMEM_SHARED((ACC_ROWS, 32), f32),
        pltpu.VMEM_SHARED((CROWS, 16), f32),
    ],
)
def _segsum_kernel(z_hbm, e_hbm, sum_hbm, cnt_hbm,
                   ebuf, gidx, didx, rows, onehot, pos2, rowidx2,
                   seme0, seme1, semg0, semg1, sems0, sems1, semc0, semc1,
                   acc, cntacc):
    c = lax.axis_index("c")
    s = lax.axis_index("s")
    seme = (seme0, seme1)
    semg = (semg0, semg1)
    sems = (sems0, sems1)
    semc = (semc0, semc1)
    tstart = s * _B_ETILE

    zero16 = jnp.zeros((16,), f32)
    one16 = jnp.ones((16,), f32)
    lane16 = jnp.arange(16, dtype=i32)

    def _zero_rows(r, _):
        for g in range(2):
            rows[0, r, pl.ds(g * 16, 16)] = zero16
        onehot[0, r, :] = zero16
        onehot[1, r, :] = zero16
        return 0

    lax.fori_loop(0, 128, _zero_rows, 0)

    def _zloop(t, _):
        q = s + 16 * t
        pltpu.sync_copy(rows.at[0], acc.at[pl.ds(q * 128, 128), :])
        return 0

    lax.fori_loop(0, _Z_PER_TILE, _zloop, 0)

    @pl.when(s < _Z_LEFT)
    def _():
        q = 16 * _Z_PER_TILE + s
        pltpu.sync_copy(rows.at[0], acc.at[pl.ds(q * 128, 128), :])

    @pl.when(s < 8)
    def _():
        pltpu.sync_copy(onehot.at[0], cntacc.at[pl.ds((16 + s) * 128, 128), :])
        pltpu.sync_copy(onehot.at[0], cntacc.at[pl.ds(s * 128, 128), :])

    @pl.when((s >= 8) & (s < 16))
    def _():
        pltpu.sync_copy(onehot.at[0], cntacc.at[pl.ds(s * 128, 128), :])

    @pl.when(s == 8)
    def _():
        pltpu.sync_copy(onehot.at[0, pl.ds(0, 64), :],
                        cntacc.at[pl.ds(3072, 64), :])

    plsc.subcore_barrier()

    def _eload_desc(t, e):
        return pltpu.make_async_copy(
            e_hbm.at[:, pl.ds(tstart + t * 512, 512)], ebuf.at[e], seme[e])

    def _fire_eload(t, e):
        pltpu.async_copy(
            e_hbm.at[:, pl.ds(tstart + t * 512, 512)], ebuf.at[e], seme[e])

    def _compute_idx(e, v, p):
        for j in range(8):
            sl = pl.ds(j * 16, 16)
            esl = pl.ds(v * 128 + j * 16, 16)
            gidx[p, sl] = ebuf[e, 0, esl] + c * N_DRUG
            didx[p, sl] = ebuf[e, 1, esl]

    def _fire_gather(p):
        pltpu.async_copy(z_hbm.at[gidx.at[p]], rows.at[p], semg[p])

    def _wait_gather(p):
        pltpu.make_async_copy(z_hbm.at[gidx.at[p]], rows.at[p],
                              semg[p]).wait()

    def _fire_scatter(p):
        pltpu.async_copy(rows.at[p], acc.at[didx.at[p]], sems[p], add=True)

    def _wait_scatter(p):
        pltpu.make_async_copy(rows.at[p], acc.at[didx.at[p]], sems[p]).wait()

    def _wait_cnt(r):
        pltpu.make_async_copy(onehot.at[r], cntacc.at[rowidx2.at[r]],
                              semc[r]).wait()

    def _count_batch(p, r, first):
        if not first:
            _wait_cnt(r)
            for j in range(8):
                sl = pl.ds(j * 16, 16)
                plsc.store_scatter(onehot.at[r], [lane16 + j * 16,
                                                  pos2[r, sl]], zero16)
        for j in range(8):
            sl = pl.ds(j * 16, 16)
            d = didx[p, sl]
            pos = d & 15
            pos2[r, sl] = pos
            rowidx2[r, sl] = lax.shift_right_logical(d, 4)
            plsc.store_scatter(onehot.at[r], [lane16 + j * 16, pos], one16)
        pltpu.async_copy(onehot.at[r], cntacc.at[rowidx2.at[r]], semc[r],
                         add=True)

    _NQ = _B_NB // 4

    def _visit(t, e, en, v, first):
        p = v % 2
        q = 1 - p
        r = v // 2

        def _advance():
            if not first or v >= 1:
                _wait_scatter(q)
            if v < 3:
                _compute_idx(e, v + 1, q)
            else:
                _eload_desc(t + 1, en).wait()
                _compute_idx(en, 0, q)
                if first:
                    _fire_eload(t + 2, e)
                else:
                    @pl.when(t + 2 < _NQ)
                    def _():
                        _fire_eload(t + 2, e)
            _fire_gather(q)

        if first:
            _advance()
        else:
            @pl.when(4 * t + v + 1 < _B_NB)
            def _():
                _advance()

        _wait_gather(p)
        _fire_scatter(p)

        @pl.when(c == p)
        def _():
            _count_batch(p, r, first)

    _fire_eload(0, 0)
    _fire_eload(1, 1)
    _eload_desc(0, 0).wait()
    _compute_idx(0, 0, 0)
    _fire_gather(0)

    for v in range(4):
        _visit(0, 0, 1, v, True)

    def _dquad(i, _):
        t = 1 + 2 * i
        for dv in range(2):
            e = (1 + dv) % 2
            for v in range(4):
                _visit(t + dv, e, 1 - e, v, False)
        return 0

    lax.fori_loop(0, (_NQ - 1) // 2, _dquad, 0)

    _wait_scatter(0)
    _wait_scatter(1)
    _wait_cnt(0)
    _wait_cnt(1)

    pltpu.sync_copy(e_hbm.at[:, pl.ds(tstart + _B_NB * 128, _B_TAIL)],
                    ebuf.at[0, :, pl.ds(0, _B_TAIL)])
    for j in range(2):
        sl = pl.ds(j * 16, 16)
        gidx[0, sl] = ebuf[0, 0, sl] + c * N_DRUG
        didx[0, sl] = ebuf[0, 1, sl]
    tsl = pl.ds(0, _B_TAIL)
    cp1 = pltpu.async_copy(z_hbm.at[gidx.at[0, tsl]],
                           rows.at[0, tsl], semg0)
    cp1.wait()
    pltpu.sync_copy(rows.at[0, tsl], acc.at[didx.at[0, tsl]], add=True)

    @pl.when(c == 0)
    def _():
        for j in range(2):
            sl = pl.ds(j * 16, 16)
            plsc.store_scatter(onehot.at[0], [lane16 + j * 16,
                                              pos2[0, sl]], zero16)
        for j in range(2):
            sl = pl.ds(j * 16, 16)
            d = didx[0, sl]
            pos = d & 15
            rowidx2[0, sl] = lax.shift_right_logical(d, 4)
            plsc.store_scatter(onehot.at[0], [lane16 + j * 16, pos], one16)
        pltpu.sync_copy(onehot.at[0, tsl],
                        cntacc.at[rowidx2.at[0, tsl]], add=True)

    plsc.subcore_barrier()

    lo_r = s * _CP_ROWS
    pltpu.sync_copy(acc.at[pl.ds(lo_r, _CP_ROWS), :],
                    sum_hbm.at[pl.ds(lo_r, _CP_ROWS), pl.ds(c * 32, 32)])

    @pl.when(s == 0)
    def _():
        pltpu.sync_copy(cntacc, cnt_hbm.at[pl.ds(c * CROWS, CROWS), :])


_D_PER_TILE = L // 32
_D_NB = _D_PER_TILE // 128
_D_TAIL = _D_PER_TILE - _D_NB * 128


@functools.partial(
    pl.kernel,
    out_type=jax.ShapeDtypeStruct((L, OUT), f32),
    mesh=plsc.VectorSubcoreMesh(**_MESH),
    compiler_params=pltpu.CompilerParams(use_tc_tiling_on_sc=True,
                                         needs_layout_passes=False),
    scratch_types=[
        pltpu.VMEM((_D_PER_TILE,), i32),
        pltpu.VMEM((_D_PER_TILE,), i32),
        pltpu.VMEM((2, 128, 2 * OUT), f32),
        pltpu.VMEM((2, 128, 2 * OUT), f32),
        pltpu.VMEM((2, 128, OUT), f32),
        pltpu.SemaphoreType.DMA,
        pltpu.SemaphoreType.DMA,
        pltpu.SemaphoreType.DMA,
        pltpu.SemaphoreType.DMA,
    ],
)
def _head_kernel(dp_hbm, pp_hbm, ls_hbm, ld_hbm, out_hbm,
                 lsbig, ldbig, dbuf, pbuf, obuf, semg0, semg1, semo0, semo1):
    c = lax.axis_index("c")
    s = lax.axis_index("s")
    w = s * 2 + c
    tbase = w * _D_PER_TILE
    semg = (semg0, semg1)
    semo = (semo0, semo1)

    pltpu.sync_copy(ls_hbm.at[pl.ds(tbase, _D_PER_TILE)], lsbig)
    pltpu.sync_copy(ld_hbm.at[pl.ds(tbase, _D_PER_TILE)], ldbig)

    def _fire(b, p):
        ioff = b * 128
        pltpu.async_copy(dp_hbm.at[lsbig.at[pl.ds(ioff, 128)]],
                         dbuf.at[p], semg[p])
        pltpu.async_copy(pp_hbm.at[ldbig.at[pl.ds(ioff, 128)]],
                         pbuf.at[p], semg[p])

    def _wait_gather(b, p):
        ioff = b * 128
        pltpu.make_async_copy(dp_hbm.at[lsbig.at[pl.ds(ioff, 128)]],
                              dbuf.at[p], semg[p]).wait()
        pltpu.make_async_copy(pp_hbm.at[ldbig.at[pl.ds(ioff, 128)]],
                              pbuf.at[p], semg[p]).wait()

    def _add(p):
        def _row(r, _):
            for g in range(4):
                sl = pl.ds(g * 16, 16)
                obuf[p, r, sl] = dbuf[p, r, sl] + pbuf[p, r, sl]
            return 0

        lax.fori_loop(0, 128, _row, 0)

    def _out_desc(b, p):
        return pltpu.make_async_copy(
            obuf.at[p], out_hbm.at[pl.ds(tbase + b * 128, 128), :], semo[p])

    _fire(0, 0)
    _fire(1, 1)

    def _pair(t2, _):
        for p in range(2):
            b = 2 * t2 + p
            _wait_gather(b, p)

            @pl.when(b >= 2)
            def _():
                _out_desc(b - 2, p).wait()

            _add(p)

            @pl.when(b + 2 < _D_NB)
            def _():
                _fire(b + 2, p)

            pltpu.async_copy(
                obuf.at[p], out_hbm.at[pl.ds(tbase + b * 128, 128), :],
                semo[p])
        return 0

    lax.fori_loop(0, _D_NB // 2, _pair, 0)

    _out_desc(_D_NB - 2, 0).wait()
    _out_desc(_D_NB - 1, 1).wait()

    toff = _D_NB * 128
    cp1 = pltpu.async_copy(dp_hbm.at[lsbig.at[pl.ds(toff, _D_TAIL)]],
                           dbuf.at[0, pl.ds(0, _D_TAIL)], semg0)
    cp2 = pltpu.async_copy(pp_hbm.at[ldbig.at[pl.ds(toff, _D_TAIL)]],
                           pbuf.at[0, pl.ds(0, _D_TAIL)], semg0)
    cp1.wait()
    cp2.wait()

    def _trow(r, _):
        for g in range(4):
            sl = pl.ds(g * 16, 16)
            obuf[0, r, sl] = dbuf[0, r, sl] + pbuf[0, r, sl]
        return 0

    lax.fori_loop(0, _D_TAIL, _trow, 0)
    pltpu.sync_copy(obuf.at[0, pl.ds(0, _D_TAIL)],
                    out_hbm.at[pl.ds(tbase + toff, _D_TAIL), :])


def _impl(x_drug, x_protein, W_l, b_l, W_r, W_lin, b_lin,
          edge_index, edge_label_index):
    eidx = edge_index.astype(i32)
    ls = edge_label_index[0].astype(i32)
    ld = edge_label_index[1].astype(i32)

    zsplit, dp128 = _proj_drug(x_drug, W_l, W_lin)
    summed, cnt_pack = _segsum_kernel(zsplit, eidx)
    pp128 = _proj_prot(summed, cnt_pack, x_protein, W_r, W_lin,
                       b_l.reshape(1, D), b_lin.reshape(1, OUT))
    return _head_kernel(dp128, pp128, ls, ld)


_impl_jitted = jax.jit(_impl)


def kernel(x_drug, x_protein, W_l, b_l, W_r, W_lin, b_lin,
           edge_index, edge_label_index):
    return _impl_jitted(x_drug, x_protein, W_l, b_l, W_r, W_lin, b_lin,
                        edge_index, edge_label_index)
